# Initial kernel scaffold; baseline (speedup 1.0000x reference)
import jax
import jax.numpy as jnp
from jax.experimental import pallas as pl


def kernel(node_states, edge_lists, qkv_W, qkv_b, out_W, out_b, W1, b1, W2, b2, g1, be1, g2, be2):
    raise NotImplementedError("write your pallas kernel here")


# speedup vs baseline: 2.0248x; 2.0248x over previous
import dataclasses
import functools

import jax
import jax.numpy as jnp
import numpy as np
from jax import lax
from jax.experimental import pallas as pl
from jax.experimental.pallas import tpu as pltpu
from jax.experimental.pallas import tpu_sc as plsc

_N = 10000
_E = 160000
_D = 256
_H = 8
_FF = 1024
_HD = _D // _H
_SCALE = float(_HD) ** -0.5

_NC, _NS, _L = 2, 16, 16
_NW = _NC * _NS
_CH = 128
_N_PAD = 10240
_E_PAD = 163840
_EPW = _E_PAD // _NW
_EPC = _E_PAD // _NS
_ZROWS = _N_PAD // _NS

_f32 = jnp.float32
_i32 = jnp.int32
_C11 = (((1,), (1,)), ((), ()))

_sc_params = pltpu.CompilerParams()
if "needs_layout_passes" in pltpu.CompilerParams.__dataclass_fields__:
    _sc_params = dataclasses.replace(_sc_params, needs_layout_passes=False)
if "use_tc_tiling_on_sc" in pltpu.CompilerParams.__dataclass_fields__:
    _sc_params = dataclasses.replace(_sc_params, use_tc_tiling_on_sc=False)

_mesh = plsc.VectorSubcoreMesh(
    core_axis_name="c", subcore_axis_name="s", num_cores=_NC, num_subcores=_NS
)

_EXPAND = np.repeat(np.eye(_H, dtype=np.float32), _HD, axis=1)


def _qkv_body(ns_ref, w_ref, b_ref, q_ref, k_ref, v_ref):
    x = lax.dot_general(ns_ref[...], w_ref[...], _C11,
                        preferred_element_type=_f32) + b_ref[...]
    q_ref[...] = x[:, 0:_D]
    k_ref[...] = x[:, _D:2 * _D]
    for j in range(4):
        v_ref[j] = x[:, 2 * _D + 64 * j:2 * _D + 64 * (j + 1)]


def _qkv_proj(ns_pad, qkv_W, qkv_b):
    blk = _N_PAD // 10
    return pl.pallas_call(
        _qkv_body,
        grid=(10,),
        in_specs=[
            pl.BlockSpec((blk, _D), lambda i: (i, 0)),
            pl.BlockSpec((3 * _D, _D), lambda i: (0, 0)),
            pl.BlockSpec((1, 3 * _D), lambda i: (0, 0)),
        ],
        out_specs=[
            pl.BlockSpec((blk, _D), lambda i: (i, 0)),
            pl.BlockSpec((blk, _D), lambda i: (i, 0)),
            pl.BlockSpec((4, blk, 64), lambda i: (0, i, 0)),
        ],
        out_shape=[
            jax.ShapeDtypeStruct((_N_PAD, _D), _f32),
            jax.ShapeDtypeStruct((_N_PAD, _D), _f32),
            jax.ShapeDtypeStruct((4, _N_PAD, 64), _f32),
        ],
    )(ns_pad, qkv_W, qkv_b)


@functools.partial(
    pl.kernel,
    out_type=(
        jax.ShapeDtypeStruct((_E_PAD, _L), _f32),
        jax.ShapeDtypeStruct((_NW, _L), _f32),
    ),
    mesh=_mesh,
    compiler_params=_sc_params,
    scratch_types=[
        pltpu.VMEM((_CH,), _i32),
        pltpu.VMEM((_CH,), _i32),
        pltpu.VMEM((_CH, _D), _f32),
        pltpu.VMEM((_CH, _D), _f32),
        pltpu.VMEM((_CH, _L), _f32),
        pltpu.VMEM((_L,), _f32),
        pltpu.SemaphoreType.DMA,
        pltpu.SemaphoreType.DMA,
    ],
)
def _edge_scores(q_hbm, k_hbm, tgt_hbm, src_hbm, sc_hbm, wmax_hbm,
                 ti, si, qv, kv, sbuf, mref, sem1, sem2):
    ci = lax.axis_index("c")
    sid = lax.axis_index("s")
    wid = sid * _NC + ci
    base = wid * _EPW
    iota = lax.iota(_i32, _L)
    mref[...] = jnp.zeros((_L,), _f32)

    @pl.loop(0, _EPW // _CH)
    def _chunk(i):
        cb = base + i * _CH
        pltpu.sync_copy(tgt_hbm.at[pl.ds(cb, _CH)], ti)
        pltpu.sync_copy(src_hbm.at[pl.ds(cb, _CH)], si)
        cq = pltpu.async_copy(q_hbm.at[ti], qv, sem1)
        ck = pltpu.async_copy(k_hbm.at[si], kv, sem2)
        cq.wait()
        ck.wait()

        @pl.loop(0, _CH)
        def _edge(e):
            svec = jnp.zeros((_L,), _f32)
            for h in range(_H):
                a = qv[e, pl.ds(h * _HD, _L)] * kv[e, pl.ds(h * _HD, _L)]
                a = a + (qv[e, pl.ds(h * _HD + _L, _L)]
                         * kv[e, pl.ds(h * _HD + _L, _L)])
                sh = jnp.sum(a) * _SCALE
                svec = jnp.where(iota == h, sh, svec)
            sbuf[e, :] = svec
            mref[...] = jnp.maximum(mref[...], svec)

        pltpu.sync_copy(sbuf, sc_hbm.at[pl.ds(cb, _CH)])

    pltpu.sync_copy(mref, wmax_hbm.at[wid])


def _make_aggregate(p):
    if p == 0:
        out_type = (jax.ShapeDtypeStruct((2, _N_PAD, 64), _f32),
                    jax.ShapeDtypeStruct((2, _N_PAD, _L), _f32))
    else:
        out_type = jax.ShapeDtypeStruct((2, _N_PAD, 64), _f32)
    scratch = [
        pltpu.VMEM((_CH,), _i32),
        pltpu.VMEM((_CH,), _i32),
        pltpu.VMEM((_CH, _L), _f32),
        pltpu.VMEM((_CH, _L), _f32),
        pltpu.VMEM((_CH, 64), _f32),
        pltpu.VMEM((_CH, 64), _f32),
        pltpu.VMEM((_NW, _L), _f32),
        pltpu.SemaphoreType.DMA,
        pltpu.SemaphoreType.DMA,
        pltpu.VMEM_SHARED((_N_PAD, 64), _f32),
    ]
    if p == 0:
        scratch.append(pltpu.VMEM_SHARED((_N_PAD, _L), _f32))

    def body(sc_hbm, tgt_hbm, src_hbm, v4_hbm, wmax_hbm, *rest):
        if p == 0:
            (acc_hbm, den_hbm, ti, si, sv, ebuf, vv, mbuf, wv, sem1, sem2,
             acc_sh, den_sh) = rest
        else:
            (acc_hbm, ti, si, sv, ebuf, vv, mbuf, wv, sem1, sem2,
             acc_sh) = rest
        ci = lax.axis_index("c")
        sid = lax.axis_index("s")

        pltpu.sync_copy(wmax_hbm, wv)
        m = wv[0, :]
        for r in range(1, _NW):
            m = jnp.maximum(m, wv[r, :])
        M = jnp.max(m)

        cols = [jnp.full((_L,), 2 * p + j // 2, _i32) + ci * 4
                for j in range(4)]

        z = jnp.zeros((_L,), _f32)

        @pl.loop(0, _CH)
        def _zrow(r):
            for j in range(4):
                mbuf[r, pl.ds(j * _L, _L)] = z
            ebuf[r, :] = z

        for t in range(_ZROWS // _CH):
            r0 = sid * _ZROWS + t * _CH
            pltpu.sync_copy(mbuf, acc_sh.at[pl.ds(r0, _CH)])
            if p == 0:
                pltpu.sync_copy(ebuf, den_sh.at[pl.ds(r0, _CH)])
        plsc.subcore_barrier()

        base = sid * _EPC

        @pl.loop(0, _EPC // _CH)
        def _chunk(i):
            cb = base + i * _CH
            pltpu.sync_copy(tgt_hbm.at[pl.ds(cb, _CH)], ti)
            pltpu.sync_copy(src_hbm.at[pl.ds(cb, _CH)], si)
            cs = pltpu.async_copy(sc_hbm.at[pl.ds(cb, _CH)], sv, sem1)
            cv = pltpu.async_copy(v4_hbm.at[2 * ci + p].at[si], vv, sem2)
            cs.wait()
            cv.wait()

            @pl.loop(0, _CH)
            def _edge(e):
                ex = jnp.exp(sv[e, :] - M)
                ebuf[e, :] = ex
                rows = jnp.full((_L,), e, _i32)
                for j in range(4):
                    w = plsc.load_gather(ebuf, [rows, cols[j]])
                    mbuf[e, pl.ds(j * _L, _L)] = vv[e, pl.ds(j * _L, _L)] * w

            if p == 0:
                pltpu.sync_copy(ebuf, den_sh.at[ti], add=True)
            pltpu.sync_copy(mbuf, acc_sh.at[ti], add=True)

        plsc.subcore_barrier()
        r0 = sid * _ZROWS
        pltpu.sync_copy(acc_sh.at[pl.ds(r0, _ZROWS)],
                        acc_hbm.at[ci].at[pl.ds(r0, _ZROWS)])
        if p == 0:
            pltpu.sync_copy(den_sh.at[pl.ds(r0, _ZROWS)],
                            den_hbm.at[ci].at[pl.ds(r0, _ZROWS)])

    return pl.kernel(body, out_type=out_type, mesh=_mesh,
                     compiler_params=_sc_params, scratch_types=scratch)


_edge_aggregate0 = _make_aggregate(0)
_edge_aggregate1 = _make_aggregate(1)


def _layernorm(x, g, b):
    mu = jnp.mean(x, axis=-1, keepdims=True)
    var = jnp.mean((x - mu) ** 2, axis=-1, keepdims=True)
    return (x - mu) / jnp.sqrt(var + 1e-5) * g + b


def _final_body(ns_ref, a0_ref, a1_ref, den_ref, exp_ref, ow_ref, ob_ref,
                w1_ref, b1_ref, w2_ref, b2_ref, g1_ref, be1_ref, g2_ref,
                be2_ref, out_ref):
    num = jnp.concatenate([a0_ref[0], a1_ref[0], a0_ref[1], a1_ref[1]],
                          axis=1)
    den8 = den_ref[0][:, 0:_H]
    div = lax.dot_general(den8, exp_ref[...], (((1,), (0,)), ((), ())),
                          preferred_element_type=_f32) + 1e-16
    attn = num / div
    attn = lax.dot_general(attn, ow_ref[...], _C11,
                           preferred_element_type=_f32) + ob_ref[...]
    x = ns_ref[...] + attn
    x = _layernorm(x, g1_ref[...], be1_ref[...])
    h1 = lax.dot_general(x, w1_ref[...], _C11,
                         preferred_element_type=_f32) + b1_ref[...]
    h1 = jnp.maximum(h1, 0.0)
    ff = lax.dot_general(h1, w2_ref[...], _C11,
                         preferred_element_type=_f32) + b2_ref[...]
    x = x + ff
    out_ref[...] = _layernorm(x, g2_ref[...], be2_ref[...])


def _final_dense(ns, acc0, acc1, den, out_W, out_b, W1, b1, W2, b2,
                 g1, be1, g2, be2):
    blk = _N // 10
    full = lambda shape: pl.BlockSpec(shape, lambda i: tuple(0 for _ in shape))
    return pl.pallas_call(
        _final_body,
        grid=(10,),
        in_specs=[
            pl.BlockSpec((blk, _D), lambda i: (i, 0)),
            pl.BlockSpec((2, blk, 64), lambda i: (0, i, 0)),
            pl.BlockSpec((2, blk, 64), lambda i: (0, i, 0)),
            pl.BlockSpec((2, blk, _L), lambda i: (0, i, 0)),
            full((_H, _D)),
            full((_D, _D)),
            full((1, _D)),
            full((_FF, _D)),
            full((1, _FF)),
            full((_D, _FF)),
            full((1, _D)),
            full((1, _D)),
            full((1, _D)),
            full((1, _D)),
            full((1, _D)),
        ],
        out_specs=pl.BlockSpec((blk, _D), lambda i: (i, 0)),
        out_shape=jax.ShapeDtypeStruct((_N, _D), _f32),
    )(ns, acc0, acc1, den, jnp.asarray(_EXPAND), out_W, out_b, W1, b1, W2, b2,
      g1, be1, g2, be2)


def kernel(node_states, edge_lists, qkv_W, qkv_b, out_W, out_b,
           W1, b1, W2, b2, g1, be1, g2, be2):
    src = edge_lists[0].astype(_i32)
    tgt = edge_lists[1].astype(_i32)
    pad = jnp.full((_E_PAD - _E,), _N_PAD - 1, _i32)
    src_p = jnp.concatenate([src, pad])
    tgt_p = jnp.concatenate([tgt, pad])
    ns_pad = jnp.zeros((_N_PAD, _D), _f32).at[:_N].set(node_states)

    q, k, v4 = _qkv_proj(ns_pad, qkv_W, qkv_b.reshape(1, -1))
    scores, wmax = _edge_scores(q, k, tgt_p, src_p)
    acc0, den = _edge_aggregate0(scores, tgt_p, src_p, v4, wmax)
    acc1 = _edge_aggregate1(scores, tgt_p, src_p, v4, wmax)
    return _final_dense(node_states, acc0, acc1, den,
                        out_W, out_b.reshape(1, -1),
                        W1, b1.reshape(1, -1), W2, b2.reshape(1, -1),
                        g1.reshape(1, -1), be1.reshape(1, -1),
                        g2.reshape(1, -1), be2.reshape(1, -1))


# speedup vs baseline: 2.8684x; 1.4167x over previous
import dataclasses
import functools

import jax
import jax.numpy as jnp
import numpy as np
from jax import lax
from jax.experimental import pallas as pl
from jax.experimental.pallas import tpu as pltpu
from jax.experimental.pallas import tpu_sc as plsc

_N = 10000
_E = 160000
_D = 256
_H = 8
_FF = 1024
_HD = _D // _H
_SCALE = float(_HD) ** -0.5

_NC, _NS, _L = 2, 16, 16
_NW = _NC * _NS
_CH = 128
_N_PAD = 10240
_E_PAD = 163840
_EPW = _E_PAD // _NW
_EPC = _E_PAD // _NS
_ZROWS = _N_PAD // _NS

_f32 = jnp.float32
_i32 = jnp.int32
_C11 = (((1,), (1,)), ((), ()))

_sc_params = pltpu.CompilerParams()
if "needs_layout_passes" in pltpu.CompilerParams.__dataclass_fields__:
    _sc_params = dataclasses.replace(_sc_params, needs_layout_passes=False)
if "use_tc_tiling_on_sc" in pltpu.CompilerParams.__dataclass_fields__:
    _sc_params = dataclasses.replace(_sc_params, use_tc_tiling_on_sc=False)

_mesh = plsc.VectorSubcoreMesh(
    core_axis_name="c", subcore_axis_name="s", num_cores=_NC, num_subcores=_NS
)

_EXPAND = np.repeat(np.eye(_H, dtype=np.float32), _HD, axis=1)


def _qkv_body(ns_ref, w_ref, b_ref, q_ref, k_ref, v_ref):
    x = lax.dot_general(ns_ref[...], w_ref[...], _C11,
                        preferred_element_type=_f32) + b_ref[...]
    q_ref[...] = x[:, 0:_D]
    k_ref[...] = x[:, _D:2 * _D]
    for j in range(4):
        v_ref[j] = x[:, 2 * _D + 64 * j:2 * _D + 64 * (j + 1)]


def _qkv_proj(ns_pad, qkv_W, qkv_b):
    blk = _N_PAD // 10
    return pl.pallas_call(
        _qkv_body,
        grid=(10,),
        in_specs=[
            pl.BlockSpec((blk, _D), lambda i: (i, 0)),
            pl.BlockSpec((3 * _D, _D), lambda i: (0, 0)),
            pl.BlockSpec((1, 3 * _D), lambda i: (0, 0)),
        ],
        out_specs=[
            pl.BlockSpec((blk, _D), lambda i: (i, 0)),
            pl.BlockSpec((blk, _D), lambda i: (i, 0)),
            pl.BlockSpec((4, blk, 64), lambda i: (0, i, 0)),
        ],
        out_shape=[
            jax.ShapeDtypeStruct((_N_PAD, _D), _f32),
            jax.ShapeDtypeStruct((_N_PAD, _D), _f32),
            jax.ShapeDtypeStruct((4, _N_PAD, 64), _f32),
        ],
    )(ns_pad, qkv_W, qkv_b)


_CHA = 64
_NCHA = _EPW // _CHA

_scores_scratch = []
for _ in range(2):
    _scores_scratch += [
        pltpu.VMEM((_CHA,), _i32),
        pltpu.VMEM((_CHA,), _i32),
        pltpu.VMEM((_CHA, _D), _f32),
        pltpu.VMEM((_CHA, _D), _f32),
        pltpu.VMEM((_CHA, _L), _f32),
        pltpu.SemaphoreType.DMA,
        pltpu.SemaphoreType.DMA,
        pltpu.SemaphoreType.DMA,
    ]
_scores_scratch.append(pltpu.VMEM((_L,), _f32))


@functools.partial(
    pl.kernel,
    out_type=(
        jax.ShapeDtypeStruct((_E_PAD, _L), _f32),
        jax.ShapeDtypeStruct((_NW, _L), _f32),
    ),
    mesh=_mesh,
    compiler_params=_sc_params,
    scratch_types=_scores_scratch,
)
def _edge_scores(q_hbm, k_hbm, tgt_hbm, src_hbm, sc_hbm, wmax_hbm, *scr):
    bufs = [scr[0:8], scr[8:16]]
    mref = scr[16]
    ci = lax.axis_index("c")
    sid = lax.axis_index("s")
    wid = sid * _NC + ci
    base = wid * _EPW
    iota = lax.iota(_i32, _L)
    mref[...] = jnp.zeros((_L,), _f32)

    def prefetch(c, b, first):
        ti, si, qv, kv, sbuf, sq, sk, so = bufs[b]
        if not first:
            pltpu.make_async_copy(sbuf, sc_hbm.at[pl.ds(base, _CHA)], so).wait()
        cb = base + c * _CHA
        pltpu.sync_copy(tgt_hbm.at[pl.ds(cb, _CHA)], ti)
        pltpu.sync_copy(src_hbm.at[pl.ds(cb, _CHA)], si)
        pltpu.async_copy(q_hbm.at[ti], qv, sq)
        pltpu.async_copy(k_hbm.at[si], kv, sk)

    def consume(c, b):
        ti, si, qv, kv, sbuf, sq, sk, so = bufs[b]
        cb = base + c * _CHA
        pltpu.make_async_copy(q_hbm.at[ti], qv, sq).wait()
        pltpu.make_async_copy(k_hbm.at[si], kv, sk).wait()

        @pl.loop(0, _CHA)
        def _edge(e):
            svec = jnp.zeros((_L,), _f32)
            for h in range(_H):
                a = qv[e, pl.ds(h * _HD, _L)] * kv[e, pl.ds(h * _HD, _L)]
                a = a + (qv[e, pl.ds(h * _HD + _L, _L)]
                         * kv[e, pl.ds(h * _HD + _L, _L)])
                sh = jnp.sum(a) * _SCALE
                svec = jnp.where(iota == h, sh, svec)
            sbuf[e, :] = svec
            mref[...] = jnp.maximum(mref[...], svec)

        pltpu.async_copy(sbuf, sc_hbm.at[pl.ds(cb, _CHA)], so)

    prefetch(0, 0, True)
    prefetch(1, 1, True)

    @pl.loop(0, _NCHA // 2)
    def _pair(g):
        c0 = 2 * g
        consume(c0, 0)

        @pl.when(g < _NCHA // 2 - 1)
        def _():
            prefetch(c0 + 2, 0, False)

        consume(c0 + 1, 1)

        @pl.when(g < _NCHA // 2 - 1)
        def _():
            prefetch(c0 + 3, 1, False)

    pltpu.make_async_copy(bufs[0][4], sc_hbm.at[pl.ds(base, _CHA)],
                          bufs[0][7]).wait()
    pltpu.make_async_copy(bufs[1][4], sc_hbm.at[pl.ds(base, _CHA)],
                          bufs[1][7]).wait()
    pltpu.sync_copy(mref, wmax_hbm.at[wid])


def _make_aggregate(p):
    if p == 0:
        out_type = (jax.ShapeDtypeStruct((2, _N_PAD, 64), _f32),
                    jax.ShapeDtypeStruct((2, _N_PAD, _L), _f32))
    else:
        out_type = jax.ShapeDtypeStruct((2, _N_PAD, 64), _f32)
    scratch = []
    for _ in range(2):
        scratch += [
            pltpu.VMEM((_CH,), _i32),
            pltpu.VMEM((_CH,), _i32),
            pltpu.VMEM((_CH, _L), _f32),
            pltpu.VMEM((_CH, _L), _f32),
            pltpu.VMEM((_CH, 64), _f32),
            pltpu.VMEM((_CH, 64), _f32),
            pltpu.SemaphoreType.DMA,
            pltpu.SemaphoreType.DMA,
            pltpu.SemaphoreType.DMA,
            pltpu.SemaphoreType.DMA,
        ]
    scratch += [
        pltpu.VMEM((_NW, _L), _f32),
        pltpu.VMEM_SHARED((_N_PAD, 64), _f32),
    ]
    if p == 0:
        scratch.append(pltpu.VMEM_SHARED((_N_PAD, _L), _f32))

    def body(sc_hbm, tgt_hbm, src_hbm, v4_hbm, wmax_hbm, *rest):
        if p == 0:
            acc_hbm, den_hbm = rest[0], rest[1]
            scr = rest[2:]
            den_sh = scr[22]
        else:
            acc_hbm = rest[0]
            scr = rest[1:]
            den_sh = None
        bufs = [scr[0:10], scr[10:20]]
        wv = scr[20]
        acc_sh = scr[21]
        ci = lax.axis_index("c")
        sid = lax.axis_index("s")

        pltpu.sync_copy(wmax_hbm, wv)
        m = wv[0, :]
        for r in range(1, _NW):
            m = jnp.maximum(m, wv[r, :])
        M = jnp.max(m)

        cols = [jnp.full((_L,), 2 * p + j // 2, _i32) + ci * 4
                for j in range(4)]

        z = jnp.zeros((_L,), _f32)
        mbuf0, ebuf0 = bufs[0][5], bufs[0][3]

        @pl.loop(0, _CH)
        def _zrow(r):
            for j in range(4):
                mbuf0[r, pl.ds(j * _L, _L)] = z
            ebuf0[r, :] = z

        for t in range(_ZROWS // _CH):
            r0 = sid * _ZROWS + t * _CH
            pltpu.sync_copy(mbuf0, acc_sh.at[pl.ds(r0, _CH)])
            if p == 0:
                pltpu.sync_copy(ebuf0, den_sh.at[pl.ds(r0, _CH)])
        plsc.subcore_barrier()

        base = sid * _EPC
        vq = v4_hbm.at[2 * ci + p]

        def prefetch(c, b, first):
            ti, si, sv, ebuf, vv, mbuf, ss, sV, sE, sM = bufs[b]
            if not first:
                if p == 0:
                    pltpu.make_async_copy(ebuf, den_sh.at[ti], sE).wait()
                pltpu.make_async_copy(mbuf, acc_sh.at[ti], sM).wait()
            cb = base + c * _CH
            pltpu.sync_copy(tgt_hbm.at[pl.ds(cb, _CH)], ti)
            pltpu.sync_copy(src_hbm.at[pl.ds(cb, _CH)], si)
            pltpu.async_copy(sc_hbm.at[pl.ds(cb, _CH)], sv, ss)
            pltpu.async_copy(vq.at[si], vv, sV)

        def consume(c, b):
            ti, si, sv, ebuf, vv, mbuf, ss, sV, sE, sM = bufs[b]
            cb = base + c * _CH
            pltpu.make_async_copy(sc_hbm.at[pl.ds(cb, _CH)], sv, ss).wait()
            pltpu.make_async_copy(vq.at[si], vv, sV).wait()

            @pl.loop(0, _CH)
            def _edge(e):
                ex = jnp.exp(sv[e, :] - M)
                ebuf[e, :] = ex
                rows = jnp.full((_L,), e, _i32)
                for j in range(4):
                    w = plsc.load_gather(ebuf, [rows, cols[j]])
                    mbuf[e, pl.ds(j * _L, _L)] = vv[e, pl.ds(j * _L, _L)] * w

            if p == 0:
                pltpu.async_copy(ebuf, den_sh.at[ti], sE, add=True)
            pltpu.async_copy(mbuf, acc_sh.at[ti], sM, add=True)

        nch = _EPC // _CH
        prefetch(0, 0, True)
        prefetch(1, 1, True)

        @pl.loop(0, nch // 2)
        def _pair(g):
            c0 = 2 * g
            consume(c0, 0)

            @pl.when(g < nch // 2 - 1)
            def _():
                prefetch(c0 + 2, 0, False)

            consume(c0 + 1, 1)

            @pl.when(g < nch // 2 - 1)
            def _():
                prefetch(c0 + 3, 1, False)

        for b in range(2):
            ti, si, sv, ebuf, vv, mbuf, ss, sV, sE, sM = bufs[b]
            if p == 0:
                pltpu.make_async_copy(ebuf, den_sh.at[ti], sE).wait()
            pltpu.make_async_copy(mbuf, acc_sh.at[ti], sM).wait()

        plsc.subcore_barrier()
        r0 = sid * _ZROWS
        pltpu.sync_copy(acc_sh.at[pl.ds(r0, _ZROWS)],
                        acc_hbm.at[ci].at[pl.ds(r0, _ZROWS)])
        if p == 0:
            pltpu.sync_copy(den_sh.at[pl.ds(r0, _ZROWS)],
                            den_hbm.at[ci].at[pl.ds(r0, _ZROWS)])

    return pl.kernel(body, out_type=out_type, mesh=_mesh,
                     compiler_params=_sc_params, scratch_types=scratch)


_edge_aggregate0 = _make_aggregate(0)
_edge_aggregate1 = _make_aggregate(1)


def _layernorm(x, g, b):
    mu = jnp.mean(x, axis=-1, keepdims=True)
    var = jnp.mean((x - mu) ** 2, axis=-1, keepdims=True)
    return (x - mu) / jnp.sqrt(var + 1e-5) * g + b


def _final_body(ns_ref, a0_ref, a1_ref, den_ref, exp_ref, ow_ref, ob_ref,
                w1_ref, b1_ref, w2_ref, b2_ref, g1_ref, be1_ref, g2_ref,
                be2_ref, out_ref):
    num = jnp.concatenate([a0_ref[0], a1_ref[0], a0_ref[1], a1_ref[1]],
                          axis=1)
    den8 = den_ref[0][:, 0:_H]
    div = lax.dot_general(den8, exp_ref[...], (((1,), (0,)), ((), ())),
                          preferred_element_type=_f32) + 1e-16
    attn = num / div
    attn = lax.dot_general(attn, ow_ref[...], _C11,
                           preferred_element_type=_f32) + ob_ref[...]
    x = ns_ref[...] + attn
    x = _layernorm(x, g1_ref[...], be1_ref[...])
    h1 = lax.dot_general(x, w1_ref[...], _C11,
                         preferred_element_type=_f32) + b1_ref[...]
    h1 = jnp.maximum(h1, 0.0)
    ff = lax.dot_general(h1, w2_ref[...], _C11,
                         preferred_element_type=_f32) + b2_ref[...]
    x = x + ff
    out_ref[...] = _layernorm(x, g2_ref[...], be2_ref[...])


def _final_dense(ns, acc0, acc1, den, out_W, out_b, W1, b1, W2, b2,
                 g1, be1, g2, be2):
    blk = _N // 10
    full = lambda shape: pl.BlockSpec(shape, lambda i: tuple(0 for _ in shape))
    return pl.pallas_call(
        _final_body,
        grid=(10,),
        in_specs=[
            pl.BlockSpec((blk, _D), lambda i: (i, 0)),
            pl.BlockSpec((2, blk, 64), lambda i: (0, i, 0)),
            pl.BlockSpec((2, blk, 64), lambda i: (0, i, 0)),
            pl.BlockSpec((2, blk, _L), lambda i: (0, i, 0)),
            full((_H, _D)),
            full((_D, _D)),
            full((1, _D)),
            full((_FF, _D)),
            full((1, _FF)),
            full((_D, _FF)),
            full((1, _D)),
            full((1, _D)),
            full((1, _D)),
            full((1, _D)),
            full((1, _D)),
        ],
        out_specs=pl.BlockSpec((blk, _D), lambda i: (i, 0)),
        out_shape=jax.ShapeDtypeStruct((_N, _D), _f32),
    )(ns, acc0, acc1, den, jnp.asarray(_EXPAND), out_W, out_b, W1, b1, W2, b2,
      g1, be1, g2, be2)


def kernel(node_states, edge_lists, qkv_W, qkv_b, out_W, out_b,
           W1, b1, W2, b2, g1, be1, g2, be2):
    src = edge_lists[0].astype(_i32)
    tgt = edge_lists[1].astype(_i32)
    pad = jnp.full((_E_PAD - _E,), _N_PAD - 1, _i32)
    src_p = jnp.concatenate([src, pad])
    tgt_p = jnp.concatenate([tgt, pad])
    ns_pad = jnp.zeros((_N_PAD, _D), _f32).at[:_N].set(node_states)

    q, k, v4 = _qkv_proj(ns_pad, qkv_W, qkv_b.reshape(1, -1))
    scores, wmax = _edge_scores(q, k, tgt_p, src_p)
    acc0, den = _edge_aggregate0(scores, tgt_p, src_p, v4, wmax)
    acc1 = _edge_aggregate1(scores, tgt_p, src_p, v4, wmax)
    return _final_dense(node_states, acc0, acc1, den,
                        out_W, out_b.reshape(1, -1),
                        W1, b1.reshape(1, -1), W2, b2.reshape(1, -1),
                        g1.reshape(1, -1), be1.reshape(1, -1),
                        g2.reshape(1, -1), be2.reshape(1, -1))


# speedup vs baseline: 3.0666x; 1.0691x over previous
import dataclasses
import functools

import jax
import jax.numpy as jnp
import numpy as np
from jax import lax
from jax.experimental import pallas as pl
from jax.experimental.pallas import tpu as pltpu
from jax.experimental.pallas import tpu_sc as plsc

_N = 10000
_E = 160000
_D = 256
_H = 8
_FF = 1024
_HD = _D // _H
_SCALE = float(_HD) ** -0.5

_NC, _NS, _L = 2, 16, 16
_NW = _NC * _NS
_CH = 128
_N_PAD = 10240
_E_PAD = 163840
_EPW = _E_PAD // _NW
_EPC = _E_PAD // _NS
_ZROWS = _N_PAD // _NS

_f32 = jnp.float32
_i32 = jnp.int32
_C11 = (((1,), (1,)), ((), ()))

_sc_params = pltpu.CompilerParams()
if "needs_layout_passes" in pltpu.CompilerParams.__dataclass_fields__:
    _sc_params = dataclasses.replace(_sc_params, needs_layout_passes=False)
if "use_tc_tiling_on_sc" in pltpu.CompilerParams.__dataclass_fields__:
    _sc_params = dataclasses.replace(_sc_params, use_tc_tiling_on_sc=False)

_mesh = plsc.VectorSubcoreMesh(
    core_axis_name="c", subcore_axis_name="s", num_cores=_NC, num_subcores=_NS
)

_EXPAND = np.repeat(np.eye(_H, dtype=np.float32), _HD, axis=1)


def _qkv_body(ns_ref, w_ref, b_ref, q_ref, k_ref, v_ref):
    x = lax.dot_general(ns_ref[...], w_ref[...], _C11,
                        preferred_element_type=_f32) + b_ref[...]
    q_ref[...] = x[:, 0:_D]
    k_ref[...] = x[:, _D:2 * _D]
    for j in range(4):
        v_ref[j] = x[:, 2 * _D + 64 * j:2 * _D + 64 * (j + 1)]


def _qkv_proj(ns_pad, qkv_W, qkv_b):
    blk = _N_PAD // 10
    return pl.pallas_call(
        _qkv_body,
        grid=(10,),
        in_specs=[
            pl.BlockSpec((blk, _D), lambda i: (i, 0)),
            pl.BlockSpec((3 * _D, _D), lambda i: (0, 0)),
            pl.BlockSpec((1, 3 * _D), lambda i: (0, 0)),
        ],
        out_specs=[
            pl.BlockSpec((blk, _D), lambda i: (i, 0)),
            pl.BlockSpec((blk, _D), lambda i: (i, 0)),
            pl.BlockSpec((4, blk, 64), lambda i: (0, i, 0)),
        ],
        out_shape=[
            jax.ShapeDtypeStruct((_N_PAD, _D), _f32),
            jax.ShapeDtypeStruct((_N_PAD, _D), _f32),
            jax.ShapeDtypeStruct((4, _N_PAD, 64), _f32),
        ],
    )(ns_pad, qkv_W, qkv_b)


_CHA = 64
_NCHA = _EPW // _CHA

_scores_scratch = []
for _ in range(2):
    _scores_scratch += [
        pltpu.VMEM((_CHA,), _i32),
        pltpu.VMEM((_CHA,), _i32),
        pltpu.VMEM((_CHA, _D), _f32),
        pltpu.VMEM((_CHA, _D), _f32),
        pltpu.VMEM((_CHA, _L), _f32),
        pltpu.SemaphoreType.DMA,
        pltpu.SemaphoreType.DMA,
        pltpu.SemaphoreType.DMA,
    ]
_scores_scratch.append(pltpu.VMEM((_L,), _f32))


@functools.partial(
    pl.kernel,
    out_type=(
        jax.ShapeDtypeStruct((_E_PAD, _L), _f32),
        jax.ShapeDtypeStruct((_NW, _L), _f32),
    ),
    mesh=_mesh,
    compiler_params=_sc_params,
    scratch_types=_scores_scratch,
)
def _edge_scores(q_hbm, k_hbm, tgt_hbm, src_hbm, sc_hbm, wmax_hbm, *scr):
    bufs = [scr[0:8], scr[8:16]]
    mref = scr[16]
    ci = lax.axis_index("c")
    sid = lax.axis_index("s")
    wid = sid * _NC + ci
    base = wid * _EPW
    iota = lax.iota(_i32, _L)
    mref[...] = jnp.zeros((_L,), _f32)

    def prefetch(c, b, first):
        ti, si, qv, kv, sbuf, sq, sk, so = bufs[b]
        if not first:
            pltpu.make_async_copy(sbuf, sc_hbm.at[pl.ds(base, _CHA)], so).wait()
        cb = base + c * _CHA
        pltpu.sync_copy(tgt_hbm.at[pl.ds(cb, _CHA)], ti)
        pltpu.sync_copy(src_hbm.at[pl.ds(cb, _CHA)], si)
        pltpu.async_copy(q_hbm.at[ti], qv, sq)
        pltpu.async_copy(k_hbm.at[si], kv, sk)

    def consume(c, b):
        ti, si, qv, kv, sbuf, sq, sk, so = bufs[b]
        cb = base + c * _CHA
        pltpu.make_async_copy(q_hbm.at[ti], qv, sq).wait()
        pltpu.make_async_copy(k_hbm.at[si], kv, sk).wait()

        @pl.loop(0, _CHA)
        def _edge(e):
            svec = jnp.zeros((_L,), _f32)
            for h in range(_H):
                a = qv[e, pl.ds(h * _HD, _L)] * kv[e, pl.ds(h * _HD, _L)]
                a = a + (qv[e, pl.ds(h * _HD + _L, _L)]
                         * kv[e, pl.ds(h * _HD + _L, _L)])
                sh = jnp.sum(a) * _SCALE
                svec = jnp.where(iota == h, sh, svec)
            sbuf[e, :] = svec
            mref[...] = jnp.maximum(mref[...], svec)

        pltpu.async_copy(sbuf, sc_hbm.at[pl.ds(cb, _CHA)], so)

    prefetch(0, 0, True)
    prefetch(1, 1, True)

    @pl.loop(0, _NCHA // 2)
    def _pair(g):
        c0 = 2 * g
        consume(c0, 0)

        @pl.when(g < _NCHA // 2 - 1)
        def _():
            prefetch(c0 + 2, 0, False)

        consume(c0 + 1, 1)

        @pl.when(g < _NCHA // 2 - 1)
        def _():
            prefetch(c0 + 3, 1, False)

    pltpu.make_async_copy(bufs[0][4], sc_hbm.at[pl.ds(base, _CHA)],
                          bufs[0][7]).wait()
    pltpu.make_async_copy(bufs[1][4], sc_hbm.at[pl.ds(base, _CHA)],
                          bufs[1][7]).wait()
    pltpu.sync_copy(mref, wmax_hbm.at[wid])


def _exp_body(wm_ref, sc_ref, ex_ref):
    M = jnp.max(wm_ref[...])
    ex_ref[...] = jnp.exp(sc_ref[...] - M)


def _exp_stage(scores, wmax):
    sc2 = scores.reshape(_E_PAD * _L // _D, _D)
    blk = sc2.shape[0] // 10
    out = pl.pallas_call(
        _exp_body,
        grid=(10,),
        in_specs=[
            pl.BlockSpec((_NW, _L), lambda i: (0, 0)),
            pl.BlockSpec((blk, _D), lambda i: (i, 0)),
        ],
        out_specs=pl.BlockSpec((blk, _D), lambda i: (i, 0)),
        out_shape=jax.ShapeDtypeStruct(sc2.shape, _f32),
    )(wmax, sc2)
    return out.reshape(_E_PAD, _L)


def _make_aggregate(p):
    if p == 0:
        out_type = (jax.ShapeDtypeStruct((2, _N_PAD, 64), _f32),
                    jax.ShapeDtypeStruct((2, _N_PAD, _L), _f32))
    else:
        out_type = jax.ShapeDtypeStruct((2, _N_PAD, 64), _f32)
    scratch = []
    for _ in range(2):
        scratch += [
            pltpu.VMEM((_CH,), _i32),
            pltpu.VMEM((_CH,), _i32),
            pltpu.VMEM((_CH, _L), _f32),
            pltpu.VMEM((_CH, _L), _f32),
            pltpu.VMEM((_CH, 64), _f32),
            pltpu.VMEM((_CH, 64), _f32),
            pltpu.SemaphoreType.DMA,
            pltpu.SemaphoreType.DMA,
            pltpu.SemaphoreType.DMA,
            pltpu.SemaphoreType.DMA,
        ]
    scratch += [
        pltpu.VMEM((_NW, _L), _f32),
        pltpu.VMEM_SHARED((_N_PAD, 64), _f32),
    ]
    if p == 0:
        scratch.append(pltpu.VMEM_SHARED((_N_PAD, _L), _f32))

    def body(sc_hbm, tgt_hbm, src_hbm, v4_hbm, wmax_hbm, *rest):
        if p == 0:
            acc_hbm, den_hbm = rest[0], rest[1]
            scr = rest[2:]
            den_sh = scr[22]
        else:
            acc_hbm = rest[0]
            scr = rest[1:]
            den_sh = None
        bufs = [scr[0:10], scr[10:20]]
        wv = scr[20]
        acc_sh = scr[21]
        ci = lax.axis_index("c")
        sid = lax.axis_index("s")

        if p == 0:
            pltpu.sync_copy(wmax_hbm, wv)
            m = wv[0, :]
            for r in range(1, _NW):
                m = jnp.maximum(m, wv[r, :])
            M = jnp.max(m)

        cols = [jnp.full((_L,), 2 * p + j // 2, _i32) + ci * 4
                for j in range(4)]

        z = jnp.zeros((_L,), _f32)
        mbuf0, ebuf0 = bufs[0][5], bufs[0][3]

        @pl.loop(0, _CH)
        def _zrow(r):
            for j in range(4):
                mbuf0[r, pl.ds(j * _L, _L)] = z
            ebuf0[r, :] = z

        for t in range(_ZROWS // _CH):
            r0 = sid * _ZROWS + t * _CH
            pltpu.sync_copy(mbuf0, acc_sh.at[pl.ds(r0, _CH)])
            if p == 0:
                pltpu.sync_copy(ebuf0, den_sh.at[pl.ds(r0, _CH)])
        plsc.subcore_barrier()

        base = sid * _EPC
        vq = v4_hbm.at[2 * ci + p]

        def prefetch(c, b, first):
            ti, si, sv, ebuf, vv, mbuf, ss, sV, sE, sM = bufs[b]
            if not first:
                if p == 0:
                    pltpu.make_async_copy(ebuf, den_sh.at[ti], sE).wait()
                pltpu.make_async_copy(mbuf, acc_sh.at[ti], sM).wait()
            cb = base + c * _CH
            pltpu.sync_copy(tgt_hbm.at[pl.ds(cb, _CH)], ti)
            pltpu.sync_copy(src_hbm.at[pl.ds(cb, _CH)], si)
            pltpu.async_copy(sc_hbm.at[pl.ds(cb, _CH)], sv, ss)
            pltpu.async_copy(vq.at[si], vv, sV)

        def consume(c, b):
            ti, si, sv, ebuf, vv, mbuf, ss, sV, sE, sM = bufs[b]
            cb = base + c * _CH
            pltpu.make_async_copy(sc_hbm.at[pl.ds(cb, _CH)], sv, ss).wait()
            pltpu.make_async_copy(vq.at[si], vv, sV).wait()

            if p == 0:
                @pl.loop(0, _CH)
                def _edge(e):
                    ebuf[e, :] = jnp.exp(sv[e, :] - M)
                    rows = jnp.full((_L,), e, _i32)
                    for j in range(4):
                        w = plsc.load_gather(ebuf, [rows, cols[j]])
                        mbuf[e, pl.ds(j * _L, _L)] = (
                            vv[e, pl.ds(j * _L, _L)] * w)

                pltpu.async_copy(ebuf, den_sh.at[ti], sE, add=True)
            else:
                @pl.loop(0, _CH)
                def _edge(e):
                    rows = jnp.full((_L,), e, _i32)
                    for j in range(4):
                        w = plsc.load_gather(sv, [rows, cols[j]])
                        mbuf[e, pl.ds(j * _L, _L)] = (
                            vv[e, pl.ds(j * _L, _L)] * w)

            pltpu.async_copy(mbuf, acc_sh.at[ti], sM, add=True)

        nch = _EPC // _CH
        prefetch(0, 0, True)
        prefetch(1, 1, True)

        @pl.loop(0, nch // 2)
        def _pair(g):
            c0 = 2 * g
            consume(c0, 0)

            @pl.when(g < nch // 2 - 1)
            def _():
                prefetch(c0 + 2, 0, False)

            consume(c0 + 1, 1)

            @pl.when(g < nch // 2 - 1)
            def _():
                prefetch(c0 + 3, 1, False)

        for b in range(2):
            ti, si, sv, ebuf, vv, mbuf, ss, sV, sE, sM = bufs[b]
            if p == 0:
                pltpu.make_async_copy(ebuf, den_sh.at[ti], sE).wait()
            pltpu.make_async_copy(mbuf, acc_sh.at[ti], sM).wait()

        plsc.subcore_barrier()
        r0 = sid * _ZROWS
        pltpu.sync_copy(acc_sh.at[pl.ds(r0, _ZROWS)],
                        acc_hbm.at[ci].at[pl.ds(r0, _ZROWS)])
        if p == 0:
            pltpu.sync_copy(den_sh.at[pl.ds(r0, _ZROWS)],
                            den_hbm.at[ci].at[pl.ds(r0, _ZROWS)])

    return pl.kernel(body, out_type=out_type, mesh=_mesh,
                     compiler_params=_sc_params, scratch_types=scratch)


_edge_aggregate0 = _make_aggregate(0)
_edge_aggregate1 = _make_aggregate(1)


def _layernorm(x, g, b):
    mu = jnp.mean(x, axis=-1, keepdims=True)
    var = jnp.mean((x - mu) ** 2, axis=-1, keepdims=True)
    return (x - mu) / jnp.sqrt(var + 1e-5) * g + b


def _final_body(ns_ref, a0_ref, a1_ref, den_ref, exp_ref, ow_ref, ob_ref,
                w1_ref, b1_ref, w2_ref, b2_ref, g1_ref, be1_ref, g2_ref,
                be2_ref, out_ref):
    num = jnp.concatenate([a0_ref[0], a1_ref[0], a0_ref[1], a1_ref[1]],
                          axis=1)
    den8 = den_ref[0][:, 0:_H]
    div = lax.dot_general(den8, exp_ref[...], (((1,), (0,)), ((), ())),
                          preferred_element_type=_f32) + 1e-16
    attn = num / div
    attn = lax.dot_general(attn, ow_ref[...], _C11,
                           preferred_element_type=_f32) + ob_ref[...]
    x = ns_ref[...] + attn
    x = _layernorm(x, g1_ref[...], be1_ref[...])
    h1 = lax.dot_general(x, w1_ref[...], _C11,
                         preferred_element_type=_f32) + b1_ref[...]
    h1 = jnp.maximum(h1, 0.0)
    ff = lax.dot_general(h1, w2_ref[...], _C11,
                         preferred_element_type=_f32) + b2_ref[...]
    x = x + ff
    out_ref[...] = _layernorm(x, g2_ref[...], be2_ref[...])


def _final_dense(ns, acc0, acc1, den, out_W, out_b, W1, b1, W2, b2,
                 g1, be1, g2, be2):
    blk = _N // 10
    full = lambda shape: pl.BlockSpec(shape, lambda i: tuple(0 for _ in shape))
    return pl.pallas_call(
        _final_body,
        grid=(10,),
        in_specs=[
            pl.BlockSpec((blk, _D), lambda i: (i, 0)),
            pl.BlockSpec((2, blk, 64), lambda i: (0, i, 0)),
            pl.BlockSpec((2, blk, 64), lambda i: (0, i, 0)),
            pl.BlockSpec((2, blk, _L), lambda i: (0, i, 0)),
            full((_H, _D)),
            full((_D, _D)),
            full((1, _D)),
            full((_FF, _D)),
            full((1, _FF)),
            full((_D, _FF)),
            full((1, _D)),
            full((1, _D)),
            full((1, _D)),
            full((1, _D)),
            full((1, _D)),
        ],
        out_specs=pl.BlockSpec((blk, _D), lambda i: (i, 0)),
        out_shape=jax.ShapeDtypeStruct((_N, _D), _f32),
    )(ns, acc0, acc1, den, jnp.asarray(_EXPAND), out_W, out_b, W1, b1, W2, b2,
      g1, be1, g2, be2)


def kernel(node_states, edge_lists, qkv_W, qkv_b, out_W, out_b,
           W1, b1, W2, b2, g1, be1, g2, be2):
    src = edge_lists[0].astype(_i32)
    tgt = edge_lists[1].astype(_i32)
    pad = jnp.full((_E_PAD - _E,), _N_PAD - 1, _i32)
    src_p = jnp.concatenate([src, pad])
    tgt_p = jnp.concatenate([tgt, pad])
    ns_pad = jnp.zeros((_N_PAD, _D), _f32).at[:_N].set(node_states)

    q, k, v4 = _qkv_proj(ns_pad, qkv_W, qkv_b.reshape(1, -1))
    scores, wmax = _edge_scores(q, k, tgt_p, src_p)
    ex = _exp_stage(scores, wmax)
    acc0, den = _edge_aggregate0(scores, tgt_p, src_p, v4, wmax)
    acc1 = _edge_aggregate1(ex, tgt_p, src_p, v4, wmax)
    return _final_dense(node_states, acc0, acc1, den,
                        out_W, out_b.reshape(1, -1),
                        W1, b1.reshape(1, -1), W2, b2.reshape(1, -1),
                        g1.reshape(1, -1), be1.reshape(1, -1),
                        g2.reshape(1, -1), be2.reshape(1, -1))


# speedup vs baseline: 3.1616x; 1.0310x over previous
import dataclasses
import functools

import jax
import jax.numpy as jnp
import numpy as np
from jax import lax
from jax.experimental import pallas as pl
from jax.experimental.pallas import tpu as pltpu
from jax.experimental.pallas import tpu_sc as plsc

_N = 10000
_E = 160000
_D = 256
_H = 8
_FF = 1024
_HD = _D // _H
_SCALE = float(_HD) ** -0.5

_NC, _NS, _L = 2, 16, 16
_NW = _NC * _NS
_CH = 128
_N_PAD = 10240
_E_PAD = 163840
_EPW = _E_PAD // _NW
_EPC = _E_PAD // _NS
_ZROWS = _N_PAD // _NS

_f32 = jnp.float32
_i32 = jnp.int32
_C11 = (((1,), (1,)), ((), ()))

_sc_params = pltpu.CompilerParams()
if "needs_layout_passes" in pltpu.CompilerParams.__dataclass_fields__:
    _sc_params = dataclasses.replace(_sc_params, needs_layout_passes=False)
if "use_tc_tiling_on_sc" in pltpu.CompilerParams.__dataclass_fields__:
    _sc_params = dataclasses.replace(_sc_params, use_tc_tiling_on_sc=False)

_mesh = plsc.VectorSubcoreMesh(
    core_axis_name="c", subcore_axis_name="s", num_cores=_NC, num_subcores=_NS
)

_EXPAND = np.repeat(np.eye(_H, dtype=np.float32), _HD, axis=1)


def _qkv_body(ns_ref, w_ref, b_ref, q_ref, k_ref, v_ref):
    x = lax.dot_general(ns_ref[...], w_ref[...], _C11,
                        preferred_element_type=_f32) + b_ref[...]
    q_ref[...] = x[:, 0:_D]
    k_ref[...] = x[:, _D:2 * _D]
    for j in range(4):
        v_ref[j] = x[:, 2 * _D + 64 * j:2 * _D + 64 * (j + 1)]


def _qkv_proj(ns_pad, qkv_W, qkv_b):
    blk = _N_PAD // 10
    return pl.pallas_call(
        _qkv_body,
        grid=(10,),
        in_specs=[
            pl.BlockSpec((blk, _D), lambda i: (i, 0)),
            pl.BlockSpec((3 * _D, _D), lambda i: (0, 0)),
            pl.BlockSpec((1, 3 * _D), lambda i: (0, 0)),
        ],
        out_specs=[
            pl.BlockSpec((blk, _D), lambda i: (i, 0)),
            pl.BlockSpec((blk, _D), lambda i: (i, 0)),
            pl.BlockSpec((4, blk, 64), lambda i: (0, i, 0)),
        ],
        out_shape=[
            jax.ShapeDtypeStruct((_N_PAD, _D), _f32),
            jax.ShapeDtypeStruct((_N_PAD, _D), _f32),
            jax.ShapeDtypeStruct((4, _N_PAD, 64), _f32),
        ],
    )(ns_pad, qkv_W, qkv_b)


_CHA = 64
_NCHA = _EPW // _CHA

_scores_scratch = []
for _ in range(2):
    _scores_scratch += [
        pltpu.VMEM((_CHA,), _i32),
        pltpu.VMEM((_CHA,), _i32),
        pltpu.VMEM((_CHA, _D), _f32),
        pltpu.VMEM((_CHA, _D), _f32),
        pltpu.VMEM((_CHA, _L), _f32),
        pltpu.SemaphoreType.DMA,
        pltpu.SemaphoreType.DMA,
        pltpu.SemaphoreType.DMA,
    ]
_scores_scratch.append(pltpu.VMEM((_L,), _f32))


@functools.partial(
    pl.kernel,
    out_type=(
        jax.ShapeDtypeStruct((_E_PAD, _L), _f32),
        jax.ShapeDtypeStruct((_NW, _L), _f32),
    ),
    mesh=_mesh,
    compiler_params=_sc_params,
    scratch_types=_scores_scratch,
)
def _edge_scores(q_hbm, k_hbm, tgt_hbm, src_hbm, sc_hbm, wmax_hbm, *scr):
    bufs = [scr[0:8], scr[8:16]]
    mref = scr[16]
    ci = lax.axis_index("c")
    sid = lax.axis_index("s")
    wid = sid * _NC + ci
    base = wid * _EPW
    iota = lax.iota(_i32, _L)
    mref[...] = jnp.zeros((_L,), _f32)

    def prefetch(c, b, first):
        ti, si, qv, kv, sbuf, sq, sk, so = bufs[b]
        if not first:
            pltpu.make_async_copy(sbuf, sc_hbm.at[pl.ds(base, _CHA)], so).wait()
        cb = base + c * _CHA
        pltpu.sync_copy(tgt_hbm.at[pl.ds(cb, _CHA)], ti)
        pltpu.sync_copy(src_hbm.at[pl.ds(cb, _CHA)], si)
        pltpu.async_copy(q_hbm.at[ti], qv, sq)
        pltpu.async_copy(k_hbm.at[si], kv, sk)

    def consume(c, b):
        ti, si, qv, kv, sbuf, sq, sk, so = bufs[b]
        cb = base + c * _CHA
        pltpu.make_async_copy(q_hbm.at[ti], qv, sq).wait()
        pltpu.make_async_copy(k_hbm.at[si], kv, sk).wait()

        @pl.loop(0, _CHA)
        def _edge(e):
            svec = jnp.zeros((_L,), _f32)
            for h in range(_H):
                a = qv[e, pl.ds(h * _HD, _L)] * kv[e, pl.ds(h * _HD, _L)]
                a = a + (qv[e, pl.ds(h * _HD + _L, _L)]
                         * kv[e, pl.ds(h * _HD + _L, _L)])
                sh = jnp.sum(a) * _SCALE
                svec = jnp.where(iota == h, sh, svec)
            sbuf[e, :] = svec
            mref[...] = jnp.maximum(mref[...], svec)

        pltpu.async_copy(sbuf, sc_hbm.at[pl.ds(cb, _CHA)], so)

    prefetch(0, 0, True)
    prefetch(1, 1, True)

    @pl.loop(0, _NCHA // 2)
    def _pair(g):
        c0 = 2 * g
        consume(c0, 0)

        @pl.when(g < _NCHA // 2 - 1)
        def _():
            prefetch(c0 + 2, 0, False)

        consume(c0 + 1, 1)

        @pl.when(g < _NCHA // 2 - 1)
        def _():
            prefetch(c0 + 3, 1, False)

    pltpu.make_async_copy(bufs[0][4], sc_hbm.at[pl.ds(base, _CHA)],
                          bufs[0][7]).wait()
    pltpu.make_async_copy(bufs[1][4], sc_hbm.at[pl.ds(base, _CHA)],
                          bufs[1][7]).wait()
    pltpu.sync_copy(mref, wmax_hbm.at[wid])


def _exp_body(wm_ref, sc_ref, ex_ref):
    M = jnp.max(wm_ref[...])
    ex_ref[...] = jnp.exp(sc_ref[...] - M)


def _exp_stage(scores, wmax):
    sc2 = scores.reshape(_E_PAD * _L // _D, _D)
    blk = sc2.shape[0] // 10
    out = pl.pallas_call(
        _exp_body,
        grid=(10,),
        in_specs=[
            pl.BlockSpec((_NW, _L), lambda i: (0, 0)),
            pl.BlockSpec((blk, _D), lambda i: (i, 0)),
        ],
        out_specs=pl.BlockSpec((blk, _D), lambda i: (i, 0)),
        out_shape=jax.ShapeDtypeStruct(sc2.shape, _f32),
    )(wmax, sc2)
    return out.reshape(_E_PAD, _L)


def _make_aggregate(p):
    aw = 80 if p == 0 else 64
    out_type = jax.ShapeDtypeStruct((2, _N_PAD, aw), _f32)
    scratch = []
    for _ in range(3):
        scratch += [
            pltpu.VMEM((_CH,), _i32),
            pltpu.VMEM((_CH,), _i32),
            pltpu.VMEM((_CH, _L), _f32),
            pltpu.VMEM((_CH, 64), _f32),
            pltpu.VMEM((_CH, aw), _f32),
            pltpu.SemaphoreType.DMA,
            pltpu.SemaphoreType.DMA,
            pltpu.SemaphoreType.DMA,
        ]
    scratch += [
        pltpu.VMEM((_NW, _L), _f32),
        pltpu.VMEM_SHARED((_N_PAD, aw), _f32),
    ]

    def body(sc_hbm, tgt_hbm, src_hbm, v4_hbm, wmax_hbm, acc_hbm, *scr):
        bufs = [scr[0:8], scr[8:16], scr[16:24]]
        wv = scr[24]
        acc_sh = scr[25]
        ci = lax.axis_index("c")
        sid = lax.axis_index("s")

        M = None
        if p == 0:
            pltpu.sync_copy(wmax_hbm, wv)
            m = wv[0, :]
            for r in range(1, _NW):
                m = jnp.maximum(m, wv[r, :])
            M = jnp.max(m)

        woff = 64 if p == 0 else 2
        cols = [jnp.full((_L,), woff + j // 2, _i32) + ci * 4
                for j in range(4)]

        z = jnp.zeros((_L,), _f32)
        mbuf0 = bufs[0][4]

        @pl.loop(0, _CH)
        def _zrow(r):
            for j in range(aw // _L):
                mbuf0[r, pl.ds(j * _L, _L)] = z

        for t in range(_ZROWS // _CH):
            r0 = sid * _ZROWS + t * _CH
            pltpu.sync_copy(mbuf0, acc_sh.at[pl.ds(r0, _CH)])
        plsc.subcore_barrier()

        base = sid * _EPC
        vq = v4_hbm.at[2 * ci + p]

        def prefetch(c, b, first):
            ti, si, sv, vv, mbuf, ss, sV, sM = bufs[b]
            if not first:
                pltpu.make_async_copy(mbuf, acc_sh.at[ti], sM).wait()
            cb = base + c * _CH
            pltpu.sync_copy(tgt_hbm.at[pl.ds(cb, _CH)], ti)
            pltpu.sync_copy(src_hbm.at[pl.ds(cb, _CH)], si)
            pltpu.async_copy(sc_hbm.at[pl.ds(cb, _CH)], sv, ss)
            pltpu.async_copy(vq.at[si], vv, sV)

        def consume(c, b):
            ti, si, sv, vv, mbuf, ss, sV, sM = bufs[b]
            cb = base + c * _CH
            pltpu.make_async_copy(sc_hbm.at[pl.ds(cb, _CH)], sv, ss).wait()
            pltpu.make_async_copy(vq.at[si], vv, sV).wait()

            if p == 0:
                @pl.loop(0, _CH)
                def _edge(e):
                    mbuf[e, pl.ds(64, _L)] = jnp.exp(sv[e, :] - M)
                    rows = jnp.full((_L,), e, _i32)
                    for j in range(4):
                        w = plsc.load_gather(mbuf, [rows, cols[j]])
                        mbuf[e, pl.ds(j * _L, _L)] = (
                            vv[e, pl.ds(j * _L, _L)] * w)
            else:
                @pl.loop(0, _CH)
                def _edge(e):
                    rows = jnp.full((_L,), e, _i32)
                    for j in range(4):
                        w = plsc.load_gather(sv, [rows, cols[j]])
                        mbuf[e, pl.ds(j * _L, _L)] = (
                            vv[e, pl.ds(j * _L, _L)] * w)

            pltpu.async_copy(mbuf, acc_sh.at[ti], sM, add=True)

        nch = _EPC // _CH
        prefetch(0, 0, True)
        prefetch(1, 1, True)
        prefetch(2, 2, True)

        @pl.loop(0, (nch + 2) // 3)
        def _trip(g):
            c = 3 * g
            consume(c, 0)

            @pl.when(c + 1 < nch)
            def _():
                consume(c + 1, 1)

            @pl.when(c + 3 < nch)
            def _():
                prefetch(c + 3, 0, False)

            @pl.when(c + 2 < nch)
            def _():
                consume(c + 2, 2)

            @pl.when(c + 4 < nch)
            def _():
                prefetch(c + 4, 1, False)

            @pl.when(c + 5 < nch)
            def _():
                prefetch(c + 5, 2, False)

        for b in range(3):
            ti, si, sv, vv, mbuf, ss, sV, sM = bufs[b]
            pltpu.make_async_copy(mbuf, acc_sh.at[ti], sM).wait()

        plsc.subcore_barrier()
        r0 = sid * _ZROWS
        pltpu.sync_copy(acc_sh.at[pl.ds(r0, _ZROWS)],
                        acc_hbm.at[ci].at[pl.ds(r0, _ZROWS)])

    return pl.kernel(body, out_type=out_type, mesh=_mesh,
                     compiler_params=_sc_params, scratch_types=scratch)


_edge_aggregate0 = _make_aggregate(0)
_edge_aggregate1 = _make_aggregate(1)


def _layernorm(x, g, b):
    mu = jnp.mean(x, axis=-1, keepdims=True)
    var = jnp.mean((x - mu) ** 2, axis=-1, keepdims=True)
    return (x - mu) / jnp.sqrt(var + 1e-5) * g + b


def _final_body(ns_ref, a0_ref, a1_ref, exp_ref, ow_ref, ob_ref,
                w1_ref, b1_ref, w2_ref, b2_ref, g1_ref, be1_ref, g2_ref,
                be2_ref, out_ref):
    num = jnp.concatenate([a0_ref[0][:, 0:64], a1_ref[0],
                           a0_ref[1][:, 0:64], a1_ref[1]], axis=1)
    den8 = a0_ref[0][:, 64:64 + _H]
    div = lax.dot_general(den8, exp_ref[...], (((1,), (0,)), ((), ())),
                          preferred_element_type=_f32) + 1e-16
    attn = num / div
    attn = lax.dot_general(attn, ow_ref[...], _C11,
                           preferred_element_type=_f32) + ob_ref[...]
    x = ns_ref[...] + attn
    x = _layernorm(x, g1_ref[...], be1_ref[...])
    h1 = lax.dot_general(x, w1_ref[...], _C11,
                         preferred_element_type=_f32) + b1_ref[...]
    h1 = jnp.maximum(h1, 0.0)
    ff = lax.dot_general(h1, w2_ref[...], _C11,
                         preferred_element_type=_f32) + b2_ref[...]
    x = x + ff
    out_ref[...] = _layernorm(x, g2_ref[...], be2_ref[...])


def _final_dense(ns, acc0, acc1, out_W, out_b, W1, b1, W2, b2,
                 g1, be1, g2, be2):
    blk = _N // 10
    full = lambda shape: pl.BlockSpec(shape, lambda i: tuple(0 for _ in shape))
    return pl.pallas_call(
        _final_body,
        grid=(10,),
        in_specs=[
            pl.BlockSpec((blk, _D), lambda i: (i, 0)),
            pl.BlockSpec((2, blk, 80), lambda i: (0, i, 0)),
            pl.BlockSpec((2, blk, 64), lambda i: (0, i, 0)),
            full((_H, _D)),
            full((_D, _D)),
            full((1, _D)),
            full((_FF, _D)),
            full((1, _FF)),
            full((_D, _FF)),
            full((1, _D)),
            full((1, _D)),
            full((1, _D)),
            full((1, _D)),
            full((1, _D)),
        ],
        out_specs=pl.BlockSpec((blk, _D), lambda i: (i, 0)),
        out_shape=jax.ShapeDtypeStruct((_N, _D), _f32),
    )(ns, acc0, acc1, jnp.asarray(_EXPAND), out_W, out_b, W1, b1, W2, b2,
      g1, be1, g2, be2)


def kernel(node_states, edge_lists, qkv_W, qkv_b, out_W, out_b,
           W1, b1, W2, b2, g1, be1, g2, be2):
    src = edge_lists[0].astype(_i32)
    tgt = edge_lists[1].astype(_i32)
    pad = jnp.full((_E_PAD - _E,), _N_PAD - 1, _i32)
    src_p = jnp.concatenate([src, pad])
    tgt_p = jnp.concatenate([tgt, pad])
    ns_pad = jnp.zeros((_N_PAD, _D), _f32).at[:_N].set(node_states)

    q, k, v4 = _qkv_proj(ns_pad, qkv_W, qkv_b.reshape(1, -1))
    scores, wmax = _edge_scores(q, k, tgt_p, src_p)
    ex = _exp_stage(scores, wmax)
    acc0 = _edge_aggregate0(scores, tgt_p, src_p, v4, wmax)
    acc1 = _edge_aggregate1(ex, tgt_p, src_p, v4, wmax)
    return _final_dense(node_states, acc0, acc1,
                        out_W, out_b.reshape(1, -1),
                        W1, b1.reshape(1, -1), W2, b2.reshape(1, -1),
                        g1.reshape(1, -1), be1.reshape(1, -1),
                        g2.reshape(1, -1), be2.reshape(1, -1))


# speedup vs baseline: 3.2570x; 1.0302x over previous
import dataclasses
import functools

import jax
import jax.numpy as jnp
import numpy as np
from jax import lax
from jax.experimental import pallas as pl
from jax.experimental.pallas import tpu as pltpu
from jax.experimental.pallas import tpu_sc as plsc

_N = 10000
_E = 160000
_D = 256
_H = 8
_FF = 1024
_HD = _D // _H
_SCALE = float(_HD) ** -0.5

_NC, _NS, _L = 2, 16, 16
_NW = _NC * _NS
_CH = 128
_N_PAD = 10240
_E_PAD = 163840
_EPW = _E_PAD // _NW
_EPC = _E_PAD // _NS
_ZROWS = _N_PAD // _NS

_f32 = jnp.float32
_i32 = jnp.int32
_C11 = (((1,), (1,)), ((), ()))

_sc_params = pltpu.CompilerParams()
if "needs_layout_passes" in pltpu.CompilerParams.__dataclass_fields__:
    _sc_params = dataclasses.replace(_sc_params, needs_layout_passes=False)
if "use_tc_tiling_on_sc" in pltpu.CompilerParams.__dataclass_fields__:
    _sc_params = dataclasses.replace(_sc_params, use_tc_tiling_on_sc=False)

_mesh = plsc.VectorSubcoreMesh(
    core_axis_name="c", subcore_axis_name="s", num_cores=_NC, num_subcores=_NS
)

_EXPAND = np.repeat(np.eye(_H, dtype=np.float32), _HD, axis=1)


def _qkv_body(ns_ref, w_ref, b_ref, q_ref, k_ref, v_ref):
    x = lax.dot_general(ns_ref[...], w_ref[...], _C11,
                        preferred_element_type=_f32) + b_ref[...]
    q_ref[...] = x[:, 0:_D].astype(jnp.bfloat16)
    k_ref[...] = x[:, _D:2 * _D].astype(jnp.bfloat16)
    for j in range(4):
        v_ref[j] = x[:, 2 * _D + 64 * j:2 * _D + 64 * (j + 1)]


def _qkv_proj(ns, qkv_W, qkv_b):
    blk = _N // 5
    return pl.pallas_call(
        _qkv_body,
        grid=(5,),
        in_specs=[
            pl.BlockSpec((blk, _D), lambda i: (i, 0)),
            pl.BlockSpec((3 * _D, _D), lambda i: (0, 0)),
            pl.BlockSpec((1, 3 * _D), lambda i: (0, 0)),
        ],
        out_specs=[
            pl.BlockSpec((blk, _D), lambda i: (i, 0)),
            pl.BlockSpec((blk, _D), lambda i: (i, 0)),
            pl.BlockSpec((4, blk, 64), lambda i: (0, i, 0)),
        ],
        out_shape=[
            jax.ShapeDtypeStruct((_N, _D), jnp.bfloat16),
            jax.ShapeDtypeStruct((_N, _D), jnp.bfloat16),
            jax.ShapeDtypeStruct((4, _N, 64), _f32),
        ],
    )(ns, qkv_W, qkv_b)


_CHA = 128
_NCHA = _EPW // _CHA
_bf16 = jnp.bfloat16

_scores_scratch = []
for _ in range(2):
    _scores_scratch += [
        pltpu.VMEM((_CHA,), _i32),
        pltpu.VMEM((_CHA,), _i32),
        pltpu.VMEM((_CHA, _D), _bf16),
        pltpu.VMEM((_CHA, _D), _bf16),
        pltpu.VMEM((_CHA * _L,), _f32),
        pltpu.SemaphoreType.DMA,
        pltpu.SemaphoreType.DMA,
        pltpu.SemaphoreType.DMA,
    ]
_scores_scratch.append(pltpu.VMEM((_L,), _f32))


@functools.partial(
    pl.kernel,
    out_type=(
        jax.ShapeDtypeStruct((_E_PAD * _L,), _f32),
        jax.ShapeDtypeStruct((_NW, _L), _f32),
    ),
    mesh=_mesh,
    compiler_params=_sc_params,
    scratch_types=_scores_scratch,
)
def _edge_scores(q_hbm, k_hbm, tgt_hbm, src_hbm, sc_hbm, wmax_hbm, *scr):
    bufs = [scr[0:8], scr[8:16]]
    mref = scr[16]
    ci = lax.axis_index("c")
    sid = lax.axis_index("s")
    wid = sid * _NC + ci
    base = wid * _EPW
    iota = lax.iota(_i32, _L)
    fmt = plsc.PackFormat.INTERLEAVED
    mref[...] = jnp.zeros((_L,), _f32)

    def prefetch(c, b, first):
        ti, si, qv, kv, sbuf, sq, sk, so = bufs[b]
        if not first:
            pltpu.make_async_copy(sbuf, sc_hbm.at[pl.ds(base * _L, _CHA * _L)],
                                  so).wait()
        cb = base + c * _CHA
        pltpu.sync_copy(tgt_hbm.at[pl.ds(cb, _CHA)], ti)
        pltpu.sync_copy(src_hbm.at[pl.ds(cb, _CHA)], si)
        pltpu.async_copy(q_hbm.at[ti], qv, sq)
        pltpu.async_copy(k_hbm.at[si], kv, sk)

    def consume(c, b):
        ti, si, qv, kv, sbuf, sq, sk, so = bufs[b]
        cb = base + c * _CHA
        pltpu.make_async_copy(q_hbm.at[ti], qv, sq).wait()
        pltpu.make_async_copy(k_hbm.at[si], kv, sk).wait()

        @pl.loop(0, _CHA)
        def _edge(e):
            svec = jnp.zeros((_L,), _f32)
            for h in range(_H):
                qa, qb = plsc.unpack(qv[e, pl.ds(h * _HD, _HD)], format=fmt)
                ka, kb = plsc.unpack(kv[e, pl.ds(h * _HD, _HD)], format=fmt)
                a = qa * ka + qb * kb
                sh = jnp.sum(a) * _SCALE
                svec = jnp.where(iota == h, sh, svec)
            sbuf[pl.ds(e * _L, _L)] = svec
            mref[...] = jnp.maximum(mref[...], svec)

        pltpu.async_copy(sbuf, sc_hbm.at[pl.ds(cb * _L, _CHA * _L)], so)

    prefetch(0, 0, True)
    prefetch(1, 1, True)

    @pl.loop(0, _NCHA // 2)
    def _pair(g):
        c0 = 2 * g
        consume(c0, 0)

        @pl.when(g < _NCHA // 2 - 1)
        def _():
            prefetch(c0 + 2, 0, False)

        consume(c0 + 1, 1)

        @pl.when(g < _NCHA // 2 - 1)
        def _():
            prefetch(c0 + 3, 1, False)

    pltpu.make_async_copy(bufs[0][4], sc_hbm.at[pl.ds(base * _L, _CHA * _L)],
                          bufs[0][7]).wait()
    pltpu.make_async_copy(bufs[1][4], sc_hbm.at[pl.ds(base * _L, _CHA * _L)],
                          bufs[1][7]).wait()
    pltpu.sync_copy(mref, wmax_hbm.at[wid])


def _exp_body(wm_ref, sc_ref, ex_ref):
    M = jnp.max(wm_ref[...])
    ex_ref[...] = jnp.exp(sc_ref[...] - M)


def _exp_stage(scores, wmax):
    rows = _E_PAD * _L // _D
    sc2 = scores.reshape(rows, _D)
    blk = rows // 10
    out = pl.pallas_call(
        _exp_body,
        grid=(10,),
        in_specs=[
            pl.BlockSpec((_NW, _L), lambda i: (0, 0)),
            pl.BlockSpec((blk, _D), lambda i: (i, 0)),
        ],
        out_specs=pl.BlockSpec((blk, _D), lambda i: (i, 0)),
        out_shape=jax.ShapeDtypeStruct((rows, _D), _f32),
    )(wmax, sc2)
    return out.reshape(_E_PAD * _L)


def _make_aggregate(p):
    aw = 80 if p == 0 else 64
    out_type = jax.ShapeDtypeStruct((2, _N_PAD, aw), _f32)
    scratch = []
    for _ in range(3):
        scratch += [
            pltpu.VMEM((_CH,), _i32),
            pltpu.VMEM((_CH,), _i32),
            pltpu.VMEM((_CH * _L,), _f32),
            pltpu.VMEM((_CH, 64), _f32),
            pltpu.VMEM((_CH, aw), _f32),
            pltpu.SemaphoreType.DMA,
            pltpu.SemaphoreType.DMA,
            pltpu.SemaphoreType.DMA,
        ]
    scratch += [
        pltpu.VMEM((_NW, _L), _f32),
        pltpu.VMEM_SHARED((_N_PAD, aw), _f32),
    ]

    def body(sc_hbm, tgt_hbm, src_hbm, v4_hbm, wmax_hbm, acc_hbm, *scr):
        bufs = [scr[0:8], scr[8:16], scr[16:24]]
        wv = scr[24]
        acc_sh = scr[25]
        ci = lax.axis_index("c")
        sid = lax.axis_index("s")

        M = None
        if p == 0:
            pltpu.sync_copy(wmax_hbm, wv)
            m = wv[0, :]
            for r in range(1, _NW):
                m = jnp.maximum(m, wv[r, :])
            M = jnp.max(m)

        woff = 64 if p == 0 else 2
        cols = [jnp.full((_L,), woff + j // 2, _i32) + ci * 4
                for j in range(4)]

        z = jnp.zeros((_L,), _f32)
        mbuf0 = bufs[0][4]

        @pl.loop(0, _CH)
        def _zrow(r):
            for j in range(aw // _L):
                mbuf0[r, pl.ds(j * _L, _L)] = z

        for t in range(_ZROWS // _CH):
            r0 = sid * _ZROWS + t * _CH
            pltpu.sync_copy(mbuf0, acc_sh.at[pl.ds(r0, _CH)])
        plsc.subcore_barrier()

        base = sid * _EPC
        vq = v4_hbm.at[2 * ci + p]

        def prefetch(c, b, first):
            ti, si, sv, vv, mbuf, ss, sV, sM = bufs[b]
            if not first:
                pltpu.make_async_copy(mbuf, acc_sh.at[ti], sM).wait()
            cb = base + c * _CH
            pltpu.sync_copy(tgt_hbm.at[pl.ds(cb, _CH)], ti)
            pltpu.sync_copy(src_hbm.at[pl.ds(cb, _CH)], si)
            pltpu.async_copy(sc_hbm.at[pl.ds(cb * _L, _CH * _L)], sv, ss)
            pltpu.async_copy(vq.at[si], vv, sV)

        def consume(c, b):
            ti, si, sv, vv, mbuf, ss, sV, sM = bufs[b]
            cb = base + c * _CH
            pltpu.make_async_copy(sc_hbm.at[pl.ds(cb * _L, _CH * _L)],
                                  sv, ss).wait()
            pltpu.make_async_copy(vq.at[si], vv, sV).wait()

            if p == 0:
                @pl.loop(0, _CH)
                def _edge(e):
                    mbuf[e, pl.ds(64, _L)] = jnp.exp(sv[pl.ds(e * _L, _L)] - M)
                    rows = jnp.full((_L,), e, _i32)
                    for j in range(4):
                        w = plsc.load_gather(mbuf, [rows, cols[j]])
                        mbuf[e, pl.ds(j * _L, _L)] = (
                            vv[e, pl.ds(j * _L, _L)] * w)
            else:
                @pl.loop(0, _CH)
                def _edge(e):
                    e16 = e * _L
                    for j in range(4):
                        w = plsc.load_gather(sv, [cols[j] + e16])
                        mbuf[e, pl.ds(j * _L, _L)] = (
                            vv[e, pl.ds(j * _L, _L)] * w)

            pltpu.async_copy(mbuf, acc_sh.at[ti], sM, add=True)

        nch = _EPC // _CH
        prefetch(0, 0, True)
        prefetch(1, 1, True)
        prefetch(2, 2, True)

        @pl.loop(0, (nch + 2) // 3)
        def _trip(g):
            c = 3 * g
            consume(c, 0)

            @pl.when(c + 1 < nch)
            def _():
                consume(c + 1, 1)

            @pl.when(c + 3 < nch)
            def _():
                prefetch(c + 3, 0, False)

            @pl.when(c + 2 < nch)
            def _():
                consume(c + 2, 2)

            @pl.when(c + 4 < nch)
            def _():
                prefetch(c + 4, 1, False)

            @pl.when(c + 5 < nch)
            def _():
                prefetch(c + 5, 2, False)

        for b in range(3):
            ti, si, sv, vv, mbuf, ss, sV, sM = bufs[b]
            pltpu.make_async_copy(mbuf, acc_sh.at[ti], sM).wait()

        plsc.subcore_barrier()
        r0 = sid * _ZROWS
        pltpu.sync_copy(acc_sh.at[pl.ds(r0, _ZROWS)],
                        acc_hbm.at[ci].at[pl.ds(r0, _ZROWS)])

    return pl.kernel(body, out_type=out_type, mesh=_mesh,
                     compiler_params=_sc_params, scratch_types=scratch)


_edge_aggregate0 = _make_aggregate(0)
_edge_aggregate1 = _make_aggregate(1)


def _layernorm(x, g, b):
    mu = jnp.mean(x, axis=-1, keepdims=True)
    var = jnp.mean((x - mu) ** 2, axis=-1, keepdims=True)
    return (x - mu) / jnp.sqrt(var + 1e-5) * g + b


def _final_body(ns_ref, a0_ref, a1_ref, exp_ref, ow_ref, ob_ref,
                w1_ref, b1_ref, w2_ref, b2_ref, g1_ref, be1_ref, g2_ref,
                be2_ref, out_ref):
    num = jnp.concatenate([a0_ref[0][:, 0:64], a1_ref[0],
                           a0_ref[1][:, 0:64], a1_ref[1]], axis=1)
    den8 = a0_ref[0][:, 64:64 + _H]
    div = lax.dot_general(den8, exp_ref[...], (((1,), (0,)), ((), ())),
                          preferred_element_type=_f32) + 1e-16
    attn = num / div
    attn = lax.dot_general(attn, ow_ref[...], _C11,
                           preferred_element_type=_f32) + ob_ref[...]
    x = ns_ref[...] + attn
    x = _layernorm(x, g1_ref[...], be1_ref[...])
    h1 = lax.dot_general(x, w1_ref[...], _C11,
                         preferred_element_type=_f32) + b1_ref[...]
    h1 = jnp.maximum(h1, 0.0)
    ff = lax.dot_general(h1, w2_ref[...], _C11,
                         preferred_element_type=_f32) + b2_ref[...]
    x = x + ff
    out_ref[...] = _layernorm(x, g2_ref[...], be2_ref[...])


def _final_dense(ns, acc0, acc1, out_W, out_b, W1, b1, W2, b2,
                 g1, be1, g2, be2):
    blk = _N // 10
    full = lambda shape: pl.BlockSpec(shape, lambda i: tuple(0 for _ in shape))
    return pl.pallas_call(
        _final_body,
        grid=(10,),
        in_specs=[
            pl.BlockSpec((blk, _D), lambda i: (i, 0)),
            pl.BlockSpec((2, blk, 80), lambda i: (0, i, 0)),
            pl.BlockSpec((2, blk, 64), lambda i: (0, i, 0)),
            full((_H, _D)),
            full((_D, _D)),
            full((1, _D)),
            full((_FF, _D)),
            full((1, _FF)),
            full((_D, _FF)),
            full((1, _D)),
            full((1, _D)),
            full((1, _D)),
            full((1, _D)),
            full((1, _D)),
        ],
        out_specs=pl.BlockSpec((blk, _D), lambda i: (i, 0)),
        out_shape=jax.ShapeDtypeStruct((_N, _D), _f32),
    )(ns, acc0, acc1, jnp.asarray(_EXPAND), out_W, out_b, W1, b1, W2, b2,
      g1, be1, g2, be2)


def kernel(node_states, edge_lists, qkv_W, qkv_b, out_W, out_b,
           W1, b1, W2, b2, g1, be1, g2, be2):
    src = edge_lists[0].astype(_i32)
    tgt = edge_lists[1].astype(_i32)
    zpad = jnp.zeros((_E_PAD - _E,), _i32)
    spad = jnp.full((_E_PAD - _E,), _N_PAD - 1, _i32)
    src_p = jnp.concatenate([src, zpad])
    tga = jnp.concatenate([tgt, zpad])
    tgs = jnp.concatenate([tgt, spad])

    q, k, v4 = _qkv_proj(node_states, qkv_W, qkv_b.reshape(1, -1))
    scores, wmax = _edge_scores(q, k, tga, src_p)
    ex = _exp_stage(scores, wmax)
    acc0 = _edge_aggregate0(scores, tgs, src_p, v4, wmax)
    acc1 = _edge_aggregate1(ex, tgs, src_p, v4, wmax)
    return _final_dense(node_states, acc0, acc1,
                        out_W, out_b.reshape(1, -1),
                        W1, b1.reshape(1, -1), W2, b2.reshape(1, -1),
                        g1.reshape(1, -1), be1.reshape(1, -1),
                        g2.reshape(1, -1), be2.reshape(1, -1))


# speedup vs baseline: 3.5410x; 1.0872x over previous
import dataclasses
import functools

import jax
import jax.numpy as jnp
import numpy as np
from jax import lax
from jax.experimental import pallas as pl
from jax.experimental.pallas import tpu as pltpu
from jax.experimental.pallas import tpu_sc as plsc

_N = 10000
_E = 160000
_D = 256
_H = 8
_FF = 1024
_HD = _D // _H
_SCALE = float(_HD) ** -0.5

_NC, _NS, _L = 2, 16, 16
_NW = _NC * _NS
_CH = 128
_N_PAD = 10240
_E_PAD = 163840
_EPW = _E_PAD // _NW
_EPC = _E_PAD // _NS
_ZROWS = _N_PAD // _NS

_f32 = jnp.float32
_i32 = jnp.int32
_C11 = (((1,), (1,)), ((), ()))

_sc_params = pltpu.CompilerParams()
if "needs_layout_passes" in pltpu.CompilerParams.__dataclass_fields__:
    _sc_params = dataclasses.replace(_sc_params, needs_layout_passes=False)
if "use_tc_tiling_on_sc" in pltpu.CompilerParams.__dataclass_fields__:
    _sc_params = dataclasses.replace(_sc_params, use_tc_tiling_on_sc=False)

_mesh = plsc.VectorSubcoreMesh(
    core_axis_name="c", subcore_axis_name="s", num_cores=_NC, num_subcores=_NS
)

_EXPAND = np.repeat(np.eye(_H, dtype=np.float32), _HD, axis=1)


def _qkv_body(ns_ref, w_ref, b_ref, q_ref, k_ref, v_ref):
    x = lax.dot_general(ns_ref[...], w_ref[...], _C11,
                        preferred_element_type=_f32) + b_ref[...]
    q_ref[...] = x[:, 0:_D].astype(jnp.bfloat16)
    k_ref[...] = x[:, _D:2 * _D].astype(jnp.bfloat16)
    for j in range(4):
        v_ref[j] = x[:, 2 * _D + 64 * j:2 * _D + 64 * (j + 1)]


def _qkv_proj(ns, qkv_W, qkv_b):
    blk = _N // 5
    return pl.pallas_call(
        _qkv_body,
        grid=(5,),
        in_specs=[
            pl.BlockSpec((blk, _D), lambda i: (i, 0)),
            pl.BlockSpec((3 * _D, _D), lambda i: (0, 0)),
            pl.BlockSpec((1, 3 * _D), lambda i: (0, 0)),
        ],
        out_specs=[
            pl.BlockSpec((blk, _D), lambda i: (i, 0)),
            pl.BlockSpec((blk, _D), lambda i: (i, 0)),
            pl.BlockSpec((4, blk, 64), lambda i: (0, i, 0)),
        ],
        out_shape=[
            jax.ShapeDtypeStruct((_N, _D), jnp.bfloat16),
            jax.ShapeDtypeStruct((_N, _D), jnp.bfloat16),
            jax.ShapeDtypeStruct((4, _N, 64), _f32),
        ],
    )(ns, qkv_W, qkv_b)


_CHA = 128
_NCHA = _EPW // _CHA
_bf16 = jnp.bfloat16

_scores_scratch = []
for _ in range(2):
    _scores_scratch += [
        pltpu.VMEM((_CHA,), _i32),
        pltpu.VMEM((_CHA,), _i32),
        pltpu.VMEM((_CHA, _D), _bf16),
        pltpu.VMEM((_CHA, _D), _bf16),
        pltpu.VMEM((_CHA * _L,), _f32),
        pltpu.SemaphoreType.DMA,
        pltpu.SemaphoreType.DMA,
        pltpu.SemaphoreType.DMA,
    ]
_scores_scratch.append(pltpu.VMEM((_L,), _f32))


@functools.partial(
    pl.kernel,
    out_type=(
        jax.ShapeDtypeStruct((_E_PAD * _L,), _f32),
        jax.ShapeDtypeStruct((_NW, _L), _f32),
    ),
    mesh=_mesh,
    compiler_params=_sc_params,
    scratch_types=_scores_scratch,
)
def _edge_scores(q_hbm, k_hbm, tgt_hbm, src_hbm, sc_hbm, wmax_hbm, *scr):
    bufs = [scr[0:8], scr[8:16]]
    mref = scr[16]
    ci = lax.axis_index("c")
    sid = lax.axis_index("s")
    wid = sid * _NC + ci
    base = wid * _EPW
    iota = lax.iota(_i32, _L)
    fmt = plsc.PackFormat.INTERLEAVED
    mref[...] = jnp.zeros((_L,), _f32)

    def prefetch(c, b, first):
        ti, si, qv, kv, sbuf, sq, sk, so = bufs[b]
        if not first:
            pltpu.make_async_copy(sbuf, sc_hbm.at[pl.ds(base * _L, _CHA * _L)],
                                  so).wait()
        cb = base + c * _CHA
        pltpu.sync_copy(tgt_hbm.at[pl.ds(cb, _CHA)], ti)
        pltpu.sync_copy(src_hbm.at[pl.ds(cb, _CHA)], si)
        pltpu.async_copy(q_hbm.at[ti], qv, sq)
        pltpu.async_copy(k_hbm.at[si], kv, sk)

    def consume(c, b):
        ti, si, qv, kv, sbuf, sq, sk, so = bufs[b]
        cb = base + c * _CHA
        pltpu.make_async_copy(q_hbm.at[ti], qv, sq).wait()
        pltpu.make_async_copy(k_hbm.at[si], kv, sk).wait()

        @pl.loop(0, _CHA)
        def _edge(e):
            svec = jnp.zeros((_L,), _f32)
            for h in range(_H):
                qa, qb = plsc.unpack(qv[e, pl.ds(h * _HD, _HD)], format=fmt)
                ka, kb = plsc.unpack(kv[e, pl.ds(h * _HD, _HD)], format=fmt)
                a = qa * ka + qb * kb
                sh = jnp.sum(a) * _SCALE
                svec = jnp.where(iota == h, sh, svec)
            sbuf[pl.ds(e * _L, _L)] = svec
            mref[...] = jnp.maximum(mref[...], svec)

        pltpu.async_copy(sbuf, sc_hbm.at[pl.ds(cb * _L, _CHA * _L)], so)

    prefetch(0, 0, True)
    prefetch(1, 1, True)

    @pl.loop(0, _NCHA // 2)
    def _pair(g):
        c0 = 2 * g
        consume(c0, 0)

        @pl.when(g < _NCHA // 2 - 1)
        def _():
            prefetch(c0 + 2, 0, False)

        consume(c0 + 1, 1)

        @pl.when(g < _NCHA // 2 - 1)
        def _():
            prefetch(c0 + 3, 1, False)

    pltpu.make_async_copy(bufs[0][4], sc_hbm.at[pl.ds(base * _L, _CHA * _L)],
                          bufs[0][7]).wait()
    pltpu.make_async_copy(bufs[1][4], sc_hbm.at[pl.ds(base * _L, _CHA * _L)],
                          bufs[1][7]).wait()
    pltpu.sync_copy(mref, wmax_hbm.at[wid])


def _exp_body(wm_ref, sc_ref, ex_ref):
    M = jnp.max(wm_ref[...])
    ex_ref[...] = jnp.exp(sc_ref[...] - M)


def _exp_stage(scores, wmax):
    rows = _E_PAD * _L // _D
    sc2 = scores.reshape(rows, _D)
    blk = rows // 10
    out = pl.pallas_call(
        _exp_body,
        grid=(10,),
        in_specs=[
            pl.BlockSpec((_NW, _L), lambda i: (0, 0)),
            pl.BlockSpec((blk, _D), lambda i: (i, 0)),
        ],
        out_specs=pl.BlockSpec((blk, _D), lambda i: (i, 0)),
        out_shape=jax.ShapeDtypeStruct((rows, _D), _f32),
    )(wmax, sc2)
    return out.reshape(_E_PAD * _L)


def _make_aggregate(p):
    aw = 80 if p == 0 else 64
    out_type = jax.ShapeDtypeStruct((2, _N_PAD, aw), _f32)
    scratch = []
    for _ in range(3):
        scratch += [
            pltpu.VMEM((_CH,), _i32),
            pltpu.VMEM((_CH,), _i32),
            pltpu.VMEM((_CH * _L,), _f32),
            pltpu.VMEM((_CH, 64), _f32),
            pltpu.VMEM((_CH, aw), _f32),
            pltpu.SemaphoreType.DMA,
            pltpu.SemaphoreType.DMA,
            pltpu.SemaphoreType.DMA,
        ]
    scratch += [
        pltpu.VMEM_SHARED((_N_PAD, aw), _f32),
    ]

    def body(sc_hbm, tgt_hbm, src_hbm, v4_hbm, acc_hbm, *scr):
        bufs = [scr[0:8], scr[8:16], scr[16:24]]
        acc_sh = scr[24]
        ci = lax.axis_index("c")
        sid = lax.axis_index("s")

        cols = [jnp.full((_L,), 2 * p + j // 2, _i32) + ci * 4
                for j in range(4)]

        z = jnp.zeros((_L,), _f32)
        mbuf0 = bufs[0][4]

        @pl.loop(0, _CH)
        def _zrow(r):
            for j in range(aw // _L):
                mbuf0[r, pl.ds(j * _L, _L)] = z

        for t in range(_ZROWS // _CH):
            r0 = sid * _ZROWS + t * _CH
            pltpu.sync_copy(mbuf0, acc_sh.at[pl.ds(r0, _CH)])
        plsc.subcore_barrier()

        base = sid * _EPC
        vq = v4_hbm.at[2 * ci + p]

        def prefetch(c, b, first):
            ti, si, sv, vv, mbuf, ss, sV, sM = bufs[b]
            if not first:
                pltpu.make_async_copy(mbuf, acc_sh.at[ti], sM).wait()
            cb = base + c * _CH
            pltpu.sync_copy(tgt_hbm.at[pl.ds(cb, _CH)], ti)
            pltpu.sync_copy(src_hbm.at[pl.ds(cb, _CH)], si)
            pltpu.async_copy(sc_hbm.at[pl.ds(cb * _L, _CH * _L)], sv, ss)
            pltpu.async_copy(vq.at[si], vv, sV)

        def consume(c, b):
            ti, si, sv, vv, mbuf, ss, sV, sM = bufs[b]
            cb = base + c * _CH
            pltpu.make_async_copy(sc_hbm.at[pl.ds(cb * _L, _CH * _L)],
                                  sv, ss).wait()
            pltpu.make_async_copy(vq.at[si], vv, sV).wait()

            @pl.loop(0, _CH)
            def _edge(e):
                e16 = e * _L
                if p == 0:
                    mbuf[e, pl.ds(64, _L)] = sv[pl.ds(e16, _L)]
                for j in range(4):
                    w = plsc.load_gather(sv, [cols[j] + e16])
                    mbuf[e, pl.ds(j * _L, _L)] = (
                        vv[e, pl.ds(j * _L, _L)] * w)

            pltpu.async_copy(mbuf, acc_sh.at[ti], sM, add=True)

        nch = _EPC // _CH
        prefetch(0, 0, True)
        prefetch(1, 1, True)
        prefetch(2, 2, True)

        @pl.loop(0, (nch + 2) // 3)
        def _trip(g):
            c = 3 * g
            consume(c, 0)

            @pl.when(c + 1 < nch)
            def _():
                consume(c + 1, 1)

            @pl.when(c + 3 < nch)
            def _():
                prefetch(c + 3, 0, False)

            @pl.when(c + 2 < nch)
            def _():
                consume(c + 2, 2)

            @pl.when(c + 4 < nch)
            def _():
                prefetch(c + 4, 1, False)

            @pl.when(c + 5 < nch)
            def _():
                prefetch(c + 5, 2, False)

        for b in range(3):
            ti, si, sv, vv, mbuf, ss, sV, sM = bufs[b]
            pltpu.make_async_copy(mbuf, acc_sh.at[ti], sM).wait()

        plsc.subcore_barrier()
        r0 = sid * _ZROWS
        pltpu.sync_copy(acc_sh.at[pl.ds(r0, _ZROWS)],
                        acc_hbm.at[ci].at[pl.ds(r0, _ZROWS)])

    return pl.kernel(body, out_type=out_type, mesh=_mesh,
                     compiler_params=_sc_params, scratch_types=scratch)


_edge_aggregate0 = _make_aggregate(0)
_edge_aggregate1 = _make_aggregate(1)


def _layernorm(x, g, b):
    mu = jnp.mean(x, axis=-1, keepdims=True)
    var = jnp.mean((x - mu) ** 2, axis=-1, keepdims=True)
    return (x - mu) / jnp.sqrt(var + 1e-5) * g + b


def _final_body(ns_ref, a0_ref, a1_ref, exp_ref, ow_ref, ob_ref,
                w1_ref, b1_ref, w2_ref, b2_ref, g1_ref, be1_ref, g2_ref,
                be2_ref, out_ref):
    num = jnp.concatenate([a0_ref[0][:, 0:64], a1_ref[0],
                           a0_ref[1][:, 0:64], a1_ref[1]], axis=1)
    den8 = a0_ref[0][:, 64:64 + _H]
    div = lax.dot_general(den8, exp_ref[...], (((1,), (0,)), ((), ())),
                          preferred_element_type=_f32) + 1e-16
    attn = num / div
    attn = lax.dot_general(attn, ow_ref[...], _C11,
                           preferred_element_type=_f32) + ob_ref[...]
    x = ns_ref[...] + attn
    x = _layernorm(x, g1_ref[...], be1_ref[...])
    h1 = lax.dot_general(x, w1_ref[...], _C11,
                         preferred_element_type=_f32) + b1_ref[...]
    h1 = jnp.maximum(h1, 0.0)
    ff = lax.dot_general(h1, w2_ref[...], _C11,
                         preferred_element_type=_f32) + b2_ref[...]
    x = x + ff
    out_ref[...] = _layernorm(x, g2_ref[...], be2_ref[...])


def _final_dense(ns, acc0, acc1, out_W, out_b, W1, b1, W2, b2,
                 g1, be1, g2, be2):
    blk = _N // 10
    full = lambda shape: pl.BlockSpec(shape, lambda i: tuple(0 for _ in shape))
    return pl.pallas_call(
        _final_body,
        grid=(10,),
        in_specs=[
            pl.BlockSpec((blk, _D), lambda i: (i, 0)),
            pl.BlockSpec((2, blk, 80), lambda i: (0, i, 0)),
            pl.BlockSpec((2, blk, 64), lambda i: (0, i, 0)),
            full((_H, _D)),
            full((_D, _D)),
            full((1, _D)),
            full((_FF, _D)),
            full((1, _FF)),
            full((_D, _FF)),
            full((1, _D)),
            full((1, _D)),
            full((1, _D)),
            full((1, _D)),
            full((1, _D)),
        ],
        out_specs=pl.BlockSpec((blk, _D), lambda i: (i, 0)),
        out_shape=jax.ShapeDtypeStruct((_N, _D), _f32),
    )(ns, acc0, acc1, jnp.asarray(_EXPAND), out_W, out_b, W1, b1, W2, b2,
      g1, be1, g2, be2)


def kernel(node_states, edge_lists, qkv_W, qkv_b, out_W, out_b,
           W1, b1, W2, b2, g1, be1, g2, be2):
    src = edge_lists[0].astype(_i32)
    tgt = edge_lists[1].astype(_i32)
    zpad = jnp.zeros((_E_PAD - _E,), _i32)
    spad = jnp.full((_E_PAD - _E,), _N_PAD - 1, _i32)
    src_p = jnp.concatenate([src, zpad])
    tga = jnp.concatenate([tgt, zpad])
    tgs = jnp.concatenate([tgt, spad])

    q, k, v4 = _qkv_proj(node_states, qkv_W, qkv_b.reshape(1, -1))
    scores, wmax = _edge_scores(q, k, tga, src_p)
    ex = _exp_stage(scores, wmax)
    acc0 = _edge_aggregate0(ex, tgs, src_p, v4)
    acc1 = _edge_aggregate1(ex, tgs, src_p, v4)
    return _final_dense(node_states, acc0, acc1,
                        out_W, out_b.reshape(1, -1),
                        W1, b1.reshape(1, -1), W2, b2.reshape(1, -1),
                        g1.reshape(1, -1), be1.reshape(1, -1),
                        g2.reshape(1, -1), be2.reshape(1, -1))


# speedup vs baseline: 3.5622x; 1.0060x over previous
import dataclasses
import functools

import jax
import jax.numpy as jnp
import numpy as np
from jax import lax
from jax.experimental import pallas as pl
from jax.experimental.pallas import tpu as pltpu
from jax.experimental.pallas import tpu_sc as plsc

_N = 10000
_E = 160000
_D = 256
_H = 8
_FF = 1024
_HD = _D // _H
_SCALE = float(_HD) ** -0.5

_NC, _NS, _L = 2, 16, 16
_NW = _NC * _NS
_CH = 128
_N_PAD = 10240
_E_PAD = 163840
_EPW = _E_PAD // _NW
_EPC = _E_PAD // _NS
_ZROWS = _N_PAD // _NS

_f32 = jnp.float32
_i32 = jnp.int32
_C11 = (((1,), (1,)), ((), ()))

_sc_params = pltpu.CompilerParams()
if "needs_layout_passes" in pltpu.CompilerParams.__dataclass_fields__:
    _sc_params = dataclasses.replace(_sc_params, needs_layout_passes=False)
if "use_tc_tiling_on_sc" in pltpu.CompilerParams.__dataclass_fields__:
    _sc_params = dataclasses.replace(_sc_params, use_tc_tiling_on_sc=False)

_mesh = plsc.VectorSubcoreMesh(
    core_axis_name="c", subcore_axis_name="s", num_cores=_NC, num_subcores=_NS
)

_EXPAND = np.repeat(np.eye(_H, dtype=np.float32), _HD, axis=1)


def _qkv_body(ns_ref, w_ref, b_ref, q_ref, k_ref, v_ref):
    x = lax.dot_general(ns_ref[...], w_ref[...], _C11,
                        preferred_element_type=_f32) + b_ref[...]
    q_ref[...] = x[:, 0:_D].astype(jnp.bfloat16)
    k_ref[...] = x[:, _D:2 * _D].astype(jnp.bfloat16)
    for j in range(4):
        v_ref[j] = x[:, 2 * _D + 64 * j:2 * _D + 64 * (j + 1)]


def _qkv_proj(ns, qkv_W, qkv_b):
    blk = _N // 5
    return pl.pallas_call(
        _qkv_body,
        grid=(5,),
        in_specs=[
            pl.BlockSpec((blk, _D), lambda i: (i, 0)),
            pl.BlockSpec((3 * _D, _D), lambda i: (0, 0)),
            pl.BlockSpec((1, 3 * _D), lambda i: (0, 0)),
        ],
        out_specs=[
            pl.BlockSpec((blk, _D), lambda i: (i, 0)),
            pl.BlockSpec((blk, _D), lambda i: (i, 0)),
            pl.BlockSpec((4, blk, 64), lambda i: (0, i, 0)),
        ],
        out_shape=[
            jax.ShapeDtypeStruct((_N, _D), jnp.bfloat16),
            jax.ShapeDtypeStruct((_N, _D), jnp.bfloat16),
            jax.ShapeDtypeStruct((4, _N, 64), _f32),
        ],
    )(ns, qkv_W, qkv_b)


_CHA = 128
_NCHA = _EPW // _CHA
_bf16 = jnp.bfloat16

_scores_scratch = []
for _ in range(2):
    _scores_scratch += [
        pltpu.VMEM((_CHA,), _i32),
        pltpu.VMEM((_CHA,), _i32),
        pltpu.VMEM((_CHA, _D), _bf16),
        pltpu.VMEM((_CHA, _D), _bf16),
        pltpu.VMEM((_CHA * _L,), _f32),
        pltpu.SemaphoreType.DMA,
        pltpu.SemaphoreType.DMA,
        pltpu.SemaphoreType.DMA,
    ]
_scores_scratch.append(pltpu.VMEM((_L,), _f32))


@functools.partial(
    pl.kernel,
    out_type=(
        jax.ShapeDtypeStruct((_E_PAD * _L,), _f32),
        jax.ShapeDtypeStruct((_NW, _L), _f32),
    ),
    mesh=_mesh,
    compiler_params=_sc_params,
    scratch_types=_scores_scratch,
)
def _edge_scores(q_hbm, k_hbm, tgt_hbm, src_hbm, sc_hbm, wmax_hbm, *scr):
    bufs = [scr[0:8], scr[8:16]]
    mref = scr[16]
    ci = lax.axis_index("c")
    sid = lax.axis_index("s")
    wid = sid * _NC + ci
    base = wid * _EPW
    iota = lax.iota(_i32, _L)
    fmt = plsc.PackFormat.INTERLEAVED
    mref[...] = jnp.zeros((_L,), _f32)

    def prefetch(c, b, first):
        ti, si, qv, kv, sbuf, sq, sk, so = bufs[b]
        if not first:
            pltpu.make_async_copy(sbuf, sc_hbm.at[pl.ds(base * _L, _CHA * _L)],
                                  so).wait()
        cb = base + c * _CHA
        pltpu.sync_copy(tgt_hbm.at[pl.ds(cb, _CHA)], ti)
        pltpu.sync_copy(src_hbm.at[pl.ds(cb, _CHA)], si)
        pltpu.async_copy(q_hbm.at[ti], qv, sq)
        pltpu.async_copy(k_hbm.at[si], kv, sk)

    def consume(c, b):
        ti, si, qv, kv, sbuf, sq, sk, so = bufs[b]
        cb = base + c * _CHA
        pltpu.make_async_copy(q_hbm.at[ti], qv, sq).wait()
        pltpu.make_async_copy(k_hbm.at[si], kv, sk).wait()

        @pl.loop(0, _CHA, unroll=4)
        def _edge(e):
            svec = jnp.zeros((_L,), _f32)
            for h in range(_H):
                qa, qb = plsc.unpack(qv[e, pl.ds(h * _HD, _HD)], format=fmt)
                ka, kb = plsc.unpack(kv[e, pl.ds(h * _HD, _HD)], format=fmt)
                a = qa * ka + qb * kb
                sh = jnp.sum(a) * _SCALE
                svec = jnp.where(iota == h, sh, svec)
            sbuf[pl.ds(e * _L, _L)] = svec
            mref[...] = jnp.maximum(mref[...], svec)

        pltpu.async_copy(sbuf, sc_hbm.at[pl.ds(cb * _L, _CHA * _L)], so)

    prefetch(0, 0, True)
    prefetch(1, 1, True)

    @pl.loop(0, _NCHA // 2)
    def _pair(g):
        c0 = 2 * g
        consume(c0, 0)

        @pl.when(g < _NCHA // 2 - 1)
        def _():
            prefetch(c0 + 2, 0, False)

        consume(c0 + 1, 1)

        @pl.when(g < _NCHA // 2 - 1)
        def _():
            prefetch(c0 + 3, 1, False)

    pltpu.make_async_copy(bufs[0][4], sc_hbm.at[pl.ds(base * _L, _CHA * _L)],
                          bufs[0][7]).wait()
    pltpu.make_async_copy(bufs[1][4], sc_hbm.at[pl.ds(base * _L, _CHA * _L)],
                          bufs[1][7]).wait()
    pltpu.sync_copy(mref, wmax_hbm.at[wid])


def _exp_body(wm_ref, sc_ref, ex_ref):
    M = jnp.max(wm_ref[...])
    ex_ref[...] = jnp.exp(sc_ref[...] - M)


def _exp_stage(scores, wmax):
    rows = _E_PAD * _L // _D
    sc2 = scores.reshape(rows, _D)
    blk = rows // 10
    out = pl.pallas_call(
        _exp_body,
        grid=(10,),
        in_specs=[
            pl.BlockSpec((_NW, _L), lambda i: (0, 0)),
            pl.BlockSpec((blk, _D), lambda i: (i, 0)),
        ],
        out_specs=pl.BlockSpec((blk, _D), lambda i: (i, 0)),
        out_shape=jax.ShapeDtypeStruct((rows, _D), _f32),
    )(wmax, sc2)
    return out.reshape(_E_PAD * _L)


def _make_aggregate(p):
    aw = 80 if p == 0 else 64
    out_type = jax.ShapeDtypeStruct((2, _N_PAD, aw), _f32)
    scratch = []
    for _ in range(3):
        scratch += [
            pltpu.VMEM((_CH,), _i32),
            pltpu.VMEM((_CH,), _i32),
            pltpu.VMEM((_CH * _L,), _f32),
            pltpu.VMEM((_CH, 64), _f32),
            pltpu.VMEM((_CH, aw), _f32),
            pltpu.SemaphoreType.DMA,
            pltpu.SemaphoreType.DMA,
            pltpu.SemaphoreType.DMA,
        ]
    scratch += [
        pltpu.VMEM_SHARED((_N_PAD, aw), _f32),
    ]

    def body(sc_hbm, tgt_hbm, src_hbm, v4_hbm, acc_hbm, *scr):
        bufs = [scr[0:8], scr[8:16], scr[16:24]]
        acc_sh = scr[24]
        ci = lax.axis_index("c")
        sid = lax.axis_index("s")

        cols = [jnp.full((_L,), 2 * p + j // 2, _i32) + ci * 4
                for j in range(4)]

        z = jnp.zeros((_L,), _f32)
        mbuf0 = bufs[0][4]

        @pl.loop(0, _CH)
        def _zrow(r):
            for j in range(aw // _L):
                mbuf0[r, pl.ds(j * _L, _L)] = z

        done = 0
        while done < _ZROWS:
            step = min(_CH, _ZROWS - done)
            src = mbuf0 if step == _CH else mbuf0.at[pl.ds(0, step)]
            pltpu.sync_copy(src, acc_sh.at[pl.ds(sid * _ZROWS + done, step)])
            done += step
        plsc.subcore_barrier()

        base = sid * _EPC
        vq = v4_hbm.at[2 * ci + p]

        def prefetch(c, b, first):
            ti, si, sv, vv, mbuf, ss, sV, sM = bufs[b]
            if not first:
                pltpu.make_async_copy(mbuf, acc_sh.at[ti], sM).wait()
            cb = base + c * _CH
            pltpu.sync_copy(tgt_hbm.at[pl.ds(cb, _CH)], ti)
            pltpu.sync_copy(src_hbm.at[pl.ds(cb, _CH)], si)
            pltpu.async_copy(sc_hbm.at[pl.ds(cb * _L, _CH * _L)], sv, ss)
            pltpu.async_copy(vq.at[si], vv, sV)

        def consume(c, b):
            ti, si, sv, vv, mbuf, ss, sV, sM = bufs[b]
            cb = base + c * _CH
            pltpu.make_async_copy(sc_hbm.at[pl.ds(cb * _L, _CH * _L)],
                                  sv, ss).wait()
            pltpu.make_async_copy(vq.at[si], vv, sV).wait()

            @pl.loop(0, _CH, unroll=4)
            def _edge(e):
                e16 = e * _L
                if p == 0:
                    mbuf[e, pl.ds(64, _L)] = sv[pl.ds(e16, _L)]
                for j in range(4):
                    w = plsc.load_gather(sv, [cols[j] + e16])
                    mbuf[e, pl.ds(j * _L, _L)] = (
                        vv[e, pl.ds(j * _L, _L)] * w)

            pltpu.async_copy(mbuf, acc_sh.at[ti], sM, add=True)

        nch = _EPC // _CH
        prefetch(0, 0, True)
        prefetch(1, 1, True)
        prefetch(2, 2, True)

        @pl.loop(0, (nch + 2) // 3)
        def _trip(g):
            c = 3 * g
            consume(c, 0)

            @pl.when(c + 1 < nch)
            def _():
                consume(c + 1, 1)

            @pl.when(c + 3 < nch)
            def _():
                prefetch(c + 3, 0, False)

            @pl.when(c + 2 < nch)
            def _():
                consume(c + 2, 2)

            @pl.when(c + 4 < nch)
            def _():
                prefetch(c + 4, 1, False)

            @pl.when(c + 5 < nch)
            def _():
                prefetch(c + 5, 2, False)

        for b in range(3):
            ti, si, sv, vv, mbuf, ss, sV, sM = bufs[b]
            pltpu.make_async_copy(mbuf, acc_sh.at[ti], sM).wait()

        plsc.subcore_barrier()
        r0 = sid * _ZROWS
        pltpu.sync_copy(acc_sh.at[pl.ds(r0, _ZROWS)],
                        acc_hbm.at[ci].at[pl.ds(r0, _ZROWS)])

    return pl.kernel(body, out_type=out_type, mesh=_mesh,
                     compiler_params=_sc_params, scratch_types=scratch)


_edge_aggregate0 = _make_aggregate(0)
_edge_aggregate1 = _make_aggregate(1)


def _layernorm(x, g, b):
    mu = jnp.mean(x, axis=-1, keepdims=True)
    var = jnp.mean((x - mu) ** 2, axis=-1, keepdims=True)
    return (x - mu) / jnp.sqrt(var + 1e-5) * g + b


def _final_body(ns_ref, a0_ref, a1_ref, exp_ref, ow_ref, ob_ref,
                w1_ref, b1_ref, w2_ref, b2_ref, g1_ref, be1_ref, g2_ref,
                be2_ref, out_ref):
    num = jnp.concatenate([a0_ref[0][:, 0:64], a1_ref[0],
                           a0_ref[1][:, 0:64], a1_ref[1]], axis=1)
    den8 = a0_ref[0][:, 64:64 + _H]
    div = lax.dot_general(den8, exp_ref[...], (((1,), (0,)), ((), ())),
                          preferred_element_type=_f32) + 1e-16
    attn = num / div
    attn = lax.dot_general(attn, ow_ref[...], _C11,
                           preferred_element_type=_f32) + ob_ref[...]
    x = ns_ref[...] + attn
    x = _layernorm(x, g1_ref[...], be1_ref[...])
    h1 = lax.dot_general(x, w1_ref[...], _C11,
                         preferred_element_type=_f32) + b1_ref[...]
    h1 = jnp.maximum(h1, 0.0)
    ff = lax.dot_general(h1, w2_ref[...], _C11,
                         preferred_element_type=_f32) + b2_ref[...]
    x = x + ff
    out_ref[...] = _layernorm(x, g2_ref[...], be2_ref[...])


def _final_dense(ns, acc0, acc1, out_W, out_b, W1, b1, W2, b2,
                 g1, be1, g2, be2):
    blk = _N // 10
    full = lambda shape: pl.BlockSpec(shape, lambda i: tuple(0 for _ in shape))
    return pl.pallas_call(
        _final_body,
        grid=(10,),
        in_specs=[
            pl.BlockSpec((blk, _D), lambda i: (i, 0)),
            pl.BlockSpec((2, blk, 80), lambda i: (0, i, 0)),
            pl.BlockSpec((2, blk, 64), lambda i: (0, i, 0)),
            full((_H, _D)),
            full((_D, _D)),
            full((1, _D)),
            full((_FF, _D)),
            full((1, _FF)),
            full((_D, _FF)),
            full((1, _D)),
            full((1, _D)),
            full((1, _D)),
            full((1, _D)),
            full((1, _D)),
        ],
        out_specs=pl.BlockSpec((blk, _D), lambda i: (i, 0)),
        out_shape=jax.ShapeDtypeStruct((_N, _D), _f32),
    )(ns, acc0, acc1, jnp.asarray(_EXPAND), out_W, out_b, W1, b1, W2, b2,
      g1, be1, g2, be2)


def kernel(node_states, edge_lists, qkv_W, qkv_b, out_W, out_b,
           W1, b1, W2, b2, g1, be1, g2, be2):
    src = edge_lists[0].astype(_i32)
    tgt = edge_lists[1].astype(_i32)
    zpad = jnp.zeros((_E_PAD - _E,), _i32)
    spad = jnp.full((_E_PAD - _E,), _N_PAD - 1, _i32)
    src_p = jnp.concatenate([src, zpad])
    tga = jnp.concatenate([tgt, zpad])
    tgs = jnp.concatenate([tgt, spad])

    q, k, v4 = _qkv_proj(node_states, qkv_W, qkv_b.reshape(1, -1))
    scores, wmax = _edge_scores(q, k, tga, src_p)
    ex = _exp_stage(scores, wmax)
    acc0 = _edge_aggregate0(ex, tgs, src_p, v4)
    acc1 = _edge_aggregate1(ex, tgs, src_p, v4)
    return _final_dense(node_states, acc0, acc1,
                        out_W, out_b.reshape(1, -1),
                        W1, b1.reshape(1, -1), W2, b2.reshape(1, -1),
                        g1.reshape(1, -1), be1.reshape(1, -1),
                        g2.reshape(1, -1), be2.reshape(1, -1))


# speedup vs baseline: 3.7253x; 1.0458x over previous
import dataclasses
import functools

import jax
import jax.numpy as jnp
import numpy as np
from jax import lax
from jax.experimental import pallas as pl
from jax.experimental.pallas import tpu as pltpu
from jax.experimental.pallas import tpu_sc as plsc

_N = 10000
_E = 160000
_D = 256
_H = 8
_FF = 1024
_HD = _D // _H
_SCALE = float(_HD) ** -0.5

_NC, _NS, _L = 2, 16, 16
_NW = _NC * _NS
_CH = 128
_N_PAD = 10240
_E_PAD = 163840
_EPW = _E_PAD // _NW
_EPC = _E_PAD // _NS
_ZROWS = _N_PAD // _NS

_f32 = jnp.float32
_i32 = jnp.int32
_C11 = (((1,), (1,)), ((), ()))

_sc_params = pltpu.CompilerParams()
if "needs_layout_passes" in pltpu.CompilerParams.__dataclass_fields__:
    _sc_params = dataclasses.replace(_sc_params, needs_layout_passes=False)
if "use_tc_tiling_on_sc" in pltpu.CompilerParams.__dataclass_fields__:
    _sc_params = dataclasses.replace(_sc_params, use_tc_tiling_on_sc=False)

_mesh = plsc.VectorSubcoreMesh(
    core_axis_name="c", subcore_axis_name="s", num_cores=_NC, num_subcores=_NS
)

_EXPAND = np.repeat(np.eye(_H, dtype=np.float32), _HD, axis=1)


def _qkv_body(ns_ref, w_ref, b_ref, q_ref, k_ref, v_ref):
    x = lax.dot_general(ns_ref[...], w_ref[...], _C11,
                        preferred_element_type=_f32) + b_ref[...]
    q_ref[...] = x[:, 0:_D].astype(jnp.bfloat16)
    k_ref[...] = x[:, _D:2 * _D].astype(jnp.bfloat16)
    for j in range(4):
        v_ref[j] = x[:, 2 * _D + 64 * j:2 * _D + 64 * (j + 1)]


def _qkv_proj(ns, qkv_W, qkv_b):
    blk = _N // 5
    return pl.pallas_call(
        _qkv_body,
        grid=(5,),
        in_specs=[
            pl.BlockSpec((blk, _D), lambda i: (i, 0)),
            pl.BlockSpec((3 * _D, _D), lambda i: (0, 0)),
            pl.BlockSpec((1, 3 * _D), lambda i: (0, 0)),
        ],
        out_specs=[
            pl.BlockSpec((blk, _D), lambda i: (i, 0)),
            pl.BlockSpec((blk, _D), lambda i: (i, 0)),
            pl.BlockSpec((4, blk, 64), lambda i: (0, i, 0)),
        ],
        out_shape=[
            jax.ShapeDtypeStruct((_N, _D), jnp.bfloat16),
            jax.ShapeDtypeStruct((_N, _D), jnp.bfloat16),
            jax.ShapeDtypeStruct((4, _N, 64), _f32),
        ],
    )(ns, qkv_W, qkv_b)


_CHA = 128
_NCHA = _EPW // _CHA
_bf16 = jnp.bfloat16

_scores_scratch = []
for _ in range(2):
    _scores_scratch += [
        pltpu.VMEM((_CHA,), _i32),
        pltpu.VMEM((_CHA,), _i32),
        pltpu.VMEM((_CHA, _D), _bf16),
        pltpu.VMEM((_CHA, _D), _bf16),
        pltpu.VMEM((_CHA * _L,), _f32),
        pltpu.SemaphoreType.DMA,
        pltpu.SemaphoreType.DMA,
        pltpu.SemaphoreType.DMA,
    ]
_scores_scratch.append(pltpu.VMEM((_L,), _f32))


@functools.partial(
    pl.kernel,
    out_type=(
        jax.ShapeDtypeStruct((_E_PAD * _L,), _f32),
        jax.ShapeDtypeStruct((_NW, _L), _f32),
    ),
    mesh=_mesh,
    compiler_params=_sc_params,
    scratch_types=_scores_scratch,
)
def _edge_scores(q_hbm, k_hbm, tgt_hbm, src_hbm, sc_hbm, wmax_hbm, *scr):
    bufs = [scr[0:8], scr[8:16]]
    mref = scr[16]
    ci = lax.axis_index("c")
    sid = lax.axis_index("s")
    wid = sid * _NC + ci
    base = wid * _EPW
    iota = lax.iota(_i32, _L)
    fmt = plsc.PackFormat.INTERLEAVED
    mref[...] = jnp.zeros((_L,), _f32)

    def prefetch(c, b, first):
        ti, si, qv, kv, sbuf, sq, sk, so = bufs[b]
        if not first:
            pltpu.make_async_copy(sbuf, sc_hbm.at[pl.ds(base * _L, _CHA * _L)],
                                  so).wait()
        cb = base + c * _CHA
        pltpu.sync_copy(tgt_hbm.at[pl.ds(cb, _CHA)], ti)
        pltpu.sync_copy(src_hbm.at[pl.ds(cb, _CHA)], si)
        pltpu.async_copy(q_hbm.at[ti], qv, sq)
        pltpu.async_copy(k_hbm.at[si], kv, sk)

    def consume(c, b):
        ti, si, qv, kv, sbuf, sq, sk, so = bufs[b]
        cb = base + c * _CHA
        pltpu.make_async_copy(q_hbm.at[ti], qv, sq).wait()
        pltpu.make_async_copy(k_hbm.at[si], kv, sk).wait()

        @pl.loop(0, _CHA, unroll=4)
        def _edge(e):
            svec = jnp.zeros((_L,), _f32)
            for h in range(_H):
                qa, qb = plsc.unpack(qv[e, pl.ds(h * _HD, _HD)], format=fmt)
                ka, kb = plsc.unpack(kv[e, pl.ds(h * _HD, _HD)], format=fmt)
                a = qa * ka + qb * kb
                sh = jnp.sum(a) * _SCALE
                svec = jnp.where(iota == h, sh, svec)
            sbuf[pl.ds(e * _L, _L)] = svec
            mref[...] = jnp.maximum(mref[...], svec)

        pltpu.async_copy(sbuf, sc_hbm.at[pl.ds(cb * _L, _CHA * _L)], so)

    prefetch(0, 0, True)
    prefetch(1, 1, True)

    @pl.loop(0, _NCHA // 2)
    def _pair(g):
        c0 = 2 * g
        consume(c0, 0)

        @pl.when(g < _NCHA // 2 - 1)
        def _():
            prefetch(c0 + 2, 0, False)

        consume(c0 + 1, 1)

        @pl.when(g < _NCHA // 2 - 1)
        def _():
            prefetch(c0 + 3, 1, False)

    pltpu.make_async_copy(bufs[0][4], sc_hbm.at[pl.ds(base * _L, _CHA * _L)],
                          bufs[0][7]).wait()
    pltpu.make_async_copy(bufs[1][4], sc_hbm.at[pl.ds(base * _L, _CHA * _L)],
                          bufs[1][7]).wait()
    pltpu.sync_copy(mref, wmax_hbm.at[wid])


def _exp_body(wm_ref, sc_ref, ex_ref):
    M = jnp.max(wm_ref[...])
    ex_ref[...] = jnp.exp(sc_ref[...] - M)


def _exp_stage(scores, wmax):
    total = _E_PAD * _L
    blk = total // 10
    return pl.pallas_call(
        _exp_body,
        grid=(10,),
        in_specs=[
            pl.BlockSpec((_NW, _L), lambda i: (0, 0)),
            pl.BlockSpec((blk,), lambda i: (i,)),
        ],
        out_specs=pl.BlockSpec((blk,), lambda i: (i,)),
        out_shape=jax.ShapeDtypeStruct((total,), _f32),
    )(wmax, scores)


def _make_aggregate(p):
    aw = 80 if p == 0 else 64
    out_type = jax.ShapeDtypeStruct((2, _N_PAD, aw), _f32)
    scratch = []
    for _ in range(3):
        scratch += [
            pltpu.VMEM((_CH,), _i32),
            pltpu.VMEM((_CH,), _i32),
            pltpu.VMEM((_CH * _L,), _f32),
            pltpu.VMEM((_CH, 64), _f32),
            pltpu.VMEM((_CH, aw), _f32),
            pltpu.SemaphoreType.DMA,
            pltpu.SemaphoreType.DMA,
            pltpu.SemaphoreType.DMA,
        ]
    scratch += [
        pltpu.VMEM_SHARED((_N_PAD, aw), _f32),
    ]

    def body(sc_hbm, tgt_hbm, src_hbm, v4_hbm, acc_hbm, *scr):
        bufs = [scr[0:8], scr[8:16], scr[16:24]]
        acc_sh = scr[24]
        ci = lax.axis_index("c")
        sid = lax.axis_index("s")

        cols = [jnp.full((_L,), 2 * p + j // 2, _i32) + ci * 4
                for j in range(4)]

        z = jnp.zeros((_L,), _f32)
        mbuf0 = bufs[0][4]

        @pl.loop(0, _CH)
        def _zrow(r):
            for j in range(aw // _L):
                mbuf0[r, pl.ds(j * _L, _L)] = z

        done = 0
        while done < _ZROWS:
            step = min(_CH, _ZROWS - done)
            src = mbuf0 if step == _CH else mbuf0.at[pl.ds(0, step)]
            pltpu.sync_copy(src, acc_sh.at[pl.ds(sid * _ZROWS + done, step)])
            done += step
        plsc.subcore_barrier()

        base = sid * _EPC
        vq = v4_hbm.at[2 * ci + p]

        def prefetch(c, b, first):
            ti, si, sv, vv, mbuf, ss, sV, sM = bufs[b]
            if not first:
                pltpu.make_async_copy(mbuf, acc_sh.at[ti], sM).wait()
            cb = base + c * _CH
            pltpu.sync_copy(tgt_hbm.at[pl.ds(cb, _CH)], ti)
            pltpu.sync_copy(src_hbm.at[pl.ds(cb, _CH)], si)
            pltpu.async_copy(sc_hbm.at[pl.ds(cb * _L, _CH * _L)], sv, ss)
            pltpu.async_copy(vq.at[si], vv, sV)

        def consume(c, b):
            ti, si, sv, vv, mbuf, ss, sV, sM = bufs[b]
            cb = base + c * _CH
            pltpu.make_async_copy(sc_hbm.at[pl.ds(cb * _L, _CH * _L)],
                                  sv, ss).wait()
            pltpu.make_async_copy(vq.at[si], vv, sV).wait()

            @pl.loop(0, _CH, unroll=4)
            def _edge(e):
                e16 = e * _L
                if p == 0:
                    mbuf[e, pl.ds(64, _L)] = sv[pl.ds(e16, _L)]
                for j in range(4):
                    w = plsc.load_gather(sv, [cols[j] + e16])
                    mbuf[e, pl.ds(j * _L, _L)] = (
                        vv[e, pl.ds(j * _L, _L)] * w)

            pltpu.async_copy(mbuf, acc_sh.at[ti], sM, add=True)

        nch = _EPC // _CH
        prefetch(0, 0, True)
        prefetch(1, 1, True)
        prefetch(2, 2, True)

        @pl.loop(0, (nch + 2) // 3)
        def _trip(g):
            c = 3 * g
            consume(c, 0)

            @pl.when(c + 1 < nch)
            def _():
                consume(c + 1, 1)

            @pl.when(c + 3 < nch)
            def _():
                prefetch(c + 3, 0, False)

            @pl.when(c + 2 < nch)
            def _():
                consume(c + 2, 2)

            @pl.when(c + 4 < nch)
            def _():
                prefetch(c + 4, 1, False)

            @pl.when(c + 5 < nch)
            def _():
                prefetch(c + 5, 2, False)

        for b in range(3):
            ti, si, sv, vv, mbuf, ss, sV, sM = bufs[b]
            pltpu.make_async_copy(mbuf, acc_sh.at[ti], sM).wait()

        plsc.subcore_barrier()
        r0 = sid * _ZROWS
        pltpu.sync_copy(acc_sh.at[pl.ds(r0, _ZROWS)],
                        acc_hbm.at[ci].at[pl.ds(r0, _ZROWS)])

    return pl.kernel(body, out_type=out_type, mesh=_mesh,
                     compiler_params=_sc_params, scratch_types=scratch)


_edge_aggregate0 = _make_aggregate(0)
_edge_aggregate1 = _make_aggregate(1)


def _layernorm(x, g, b):
    mu = jnp.mean(x, axis=-1, keepdims=True)
    var = jnp.mean((x - mu) ** 2, axis=-1, keepdims=True)
    return (x - mu) / jnp.sqrt(var + 1e-5) * g + b


def _final_body(ns_ref, a0_ref, a1_ref, exp_ref, ow_ref, ob_ref,
                w1_ref, b1_ref, w2_ref, b2_ref, g1_ref, be1_ref, g2_ref,
                be2_ref, out_ref):
    num = jnp.concatenate([a0_ref[0][:, 0:64], a1_ref[0],
                           a0_ref[1][:, 0:64], a1_ref[1]], axis=1)
    den8 = a0_ref[0][:, 64:64 + _H]
    div = lax.dot_general(den8, exp_ref[...], (((1,), (0,)), ((), ())),
                          preferred_element_type=_f32) + 1e-16
    attn = num / div
    attn = lax.dot_general(attn, ow_ref[...], _C11,
                           preferred_element_type=_f32) + ob_ref[...]
    x = ns_ref[...] + attn
    x = _layernorm(x, g1_ref[...], be1_ref[...])
    h1 = lax.dot_general(x, w1_ref[...], _C11,
                         preferred_element_type=_f32) + b1_ref[...]
    h1 = jnp.maximum(h1, 0.0)
    ff = lax.dot_general(h1, w2_ref[...], _C11,
                         preferred_element_type=_f32) + b2_ref[...]
    x = x + ff
    out_ref[...] = _layernorm(x, g2_ref[...], be2_ref[...])


def _final_dense(ns, acc0, acc1, out_W, out_b, W1, b1, W2, b2,
                 g1, be1, g2, be2):
    blk = _N // 10
    full = lambda shape: pl.BlockSpec(shape, lambda i: tuple(0 for _ in shape))
    return pl.pallas_call(
        _final_body,
        grid=(10,),
        in_specs=[
            pl.BlockSpec((blk, _D), lambda i: (i, 0)),
            pl.BlockSpec((2, blk, 80), lambda i: (0, i, 0)),
            pl.BlockSpec((2, blk, 64), lambda i: (0, i, 0)),
            full((_H, _D)),
            full((_D, _D)),
            full((1, _D)),
            full((_FF, _D)),
            full((1, _FF)),
            full((_D, _FF)),
            full((1, _D)),
            full((1, _D)),
            full((1, _D)),
            full((1, _D)),
            full((1, _D)),
        ],
        out_specs=pl.BlockSpec((blk, _D), lambda i: (i, 0)),
        out_shape=jax.ShapeDtypeStruct((_N, _D), _f32),
    )(ns, acc0, acc1, jnp.asarray(_EXPAND), out_W, out_b, W1, b1, W2, b2,
      g1, be1, g2, be2)


def kernel(node_states, edge_lists, qkv_W, qkv_b, out_W, out_b,
           W1, b1, W2, b2, g1, be1, g2, be2):
    src = edge_lists[0].astype(_i32)
    tgt = edge_lists[1].astype(_i32)
    zpad = jnp.zeros((_E_PAD - _E,), _i32)
    spad = jnp.full((_E_PAD - _E,), _N_PAD - 1, _i32)
    src_p = jnp.concatenate([src, zpad])
    tga = jnp.concatenate([tgt, zpad])
    tgs = jnp.concatenate([tgt, spad])

    q, k, v4 = _qkv_proj(node_states, qkv_W, qkv_b.reshape(1, -1))
    scores, wmax = _edge_scores(q, k, tga, src_p)
    ex = _exp_stage(scores, wmax)
    acc0 = _edge_aggregate0(ex, tgs, src_p, v4)
    acc1 = _edge_aggregate1(ex, tgs, src_p, v4)
    return _final_dense(node_states, acc0, acc1,
                        out_W, out_b.reshape(1, -1),
                        W1, b1.reshape(1, -1), W2, b2.reshape(1, -1),
                        g1.reshape(1, -1), be1.reshape(1, -1),
                        g2.reshape(1, -1), be2.reshape(1, -1))


# speedup vs baseline: 3.7286x; 1.0009x over previous
import dataclasses
import functools

import jax
import jax.numpy as jnp
import numpy as np
from jax import lax
from jax.experimental import pallas as pl
from jax.experimental.pallas import tpu as pltpu
from jax.experimental.pallas import tpu_sc as plsc

_N = 10000
_E = 160000
_D = 256
_H = 8
_FF = 1024
_HD = _D // _H
_SCALE = float(_HD) ** -0.5

_NC, _NS, _L = 2, 16, 16
_NW = _NC * _NS
_CH = 128
_N_PAD = 10240
_E_PAD = 163840
_EPW = _E_PAD // _NW
_EPC = _E_PAD // _NS
_ZROWS = _N_PAD // _NS

_f32 = jnp.float32
_i32 = jnp.int32
_C11 = (((1,), (1,)), ((), ()))

_sc_params = pltpu.CompilerParams()
if "needs_layout_passes" in pltpu.CompilerParams.__dataclass_fields__:
    _sc_params = dataclasses.replace(_sc_params, needs_layout_passes=False)
if "use_tc_tiling_on_sc" in pltpu.CompilerParams.__dataclass_fields__:
    _sc_params = dataclasses.replace(_sc_params, use_tc_tiling_on_sc=False)

_mesh = plsc.VectorSubcoreMesh(
    core_axis_name="c", subcore_axis_name="s", num_cores=_NC, num_subcores=_NS
)

_EXPAND = np.repeat(np.eye(_H, dtype=np.float32), _HD, axis=1)


def _qkv_body(ns_ref, w_ref, b_ref, q_ref, k_ref, v_ref):
    x = lax.dot_general(ns_ref[...], w_ref[...], _C11,
                        preferred_element_type=_f32) + b_ref[...]
    q_ref[...] = x[:, 0:_D].astype(jnp.bfloat16)
    k_ref[...] = x[:, _D:2 * _D].astype(jnp.bfloat16)
    for j in range(4):
        v_ref[j] = x[:, 2 * _D + 64 * j:2 * _D + 64 * (j + 1)]


def _qkv_proj(ns, qkv_W, qkv_b):
    blk = _N // 5
    return pl.pallas_call(
        _qkv_body,
        grid=(5,),
        in_specs=[
            pl.BlockSpec((blk, _D), lambda i: (i, 0)),
            pl.BlockSpec((3 * _D, _D), lambda i: (0, 0)),
            pl.BlockSpec((1, 3 * _D), lambda i: (0, 0)),
        ],
        out_specs=[
            pl.BlockSpec((blk, _D), lambda i: (i, 0)),
            pl.BlockSpec((blk, _D), lambda i: (i, 0)),
            pl.BlockSpec((4, blk, 64), lambda i: (0, i, 0)),
        ],
        out_shape=[
            jax.ShapeDtypeStruct((_N, _D), jnp.bfloat16),
            jax.ShapeDtypeStruct((_N, _D), jnp.bfloat16),
            jax.ShapeDtypeStruct((4, _N, 64), _f32),
        ],
    )(ns, qkv_W, qkv_b)


_CHA = 128
_NCHA = _EPW // _CHA
_bf16 = jnp.bfloat16

_scores_scratch = []
for _ in range(2):
    _scores_scratch += [
        pltpu.VMEM((_CHA,), _i32),
        pltpu.VMEM((_CHA,), _i32),
        pltpu.VMEM((_CHA, _D), _bf16),
        pltpu.VMEM((_CHA, _D), _bf16),
        pltpu.VMEM((_CHA * _L,), _f32),
        pltpu.SemaphoreType.DMA,
        pltpu.SemaphoreType.DMA,
        pltpu.SemaphoreType.DMA,
    ]
_scores_scratch.append(pltpu.VMEM((_L,), _f32))


@functools.partial(
    pl.kernel,
    out_type=(
        jax.ShapeDtypeStruct((_E_PAD * _L,), _f32),
        jax.ShapeDtypeStruct((_NW, _L), _f32),
    ),
    mesh=_mesh,
    compiler_params=_sc_params,
    scratch_types=_scores_scratch,
)
def _edge_scores(q_hbm, k_hbm, tgt_hbm, src_hbm, sc_hbm, wmax_hbm, *scr):
    bufs = [scr[0:8], scr[8:16]]
    mref = scr[16]
    ci = lax.axis_index("c")
    sid = lax.axis_index("s")
    wid = sid * _NC + ci
    base = wid * _EPW
    iota = lax.iota(_i32, _L)
    fmt = plsc.PackFormat.INTERLEAVED
    mref[...] = jnp.zeros((_L,), _f32)

    def prefetch(c, b, first):
        ti, si, qv, kv, sbuf, sq, sk, so = bufs[b]
        if not first:
            pltpu.make_async_copy(sbuf, sc_hbm.at[pl.ds(base * _L, _CHA * _L)],
                                  so).wait()
        cb = base + c * _CHA
        pltpu.sync_copy(tgt_hbm.at[pl.ds(cb, _CHA)], ti)
        pltpu.sync_copy(src_hbm.at[pl.ds(cb, _CHA)], si)
        pltpu.async_copy(q_hbm.at[ti], qv, sq)
        pltpu.async_copy(k_hbm.at[si], kv, sk)

    def consume(c, b):
        ti, si, qv, kv, sbuf, sq, sk, so = bufs[b]
        cb = base + c * _CHA
        pltpu.make_async_copy(q_hbm.at[ti], qv, sq).wait()
        pltpu.make_async_copy(k_hbm.at[si], kv, sk).wait()

        @pl.loop(0, _CHA, unroll=4)
        def _edge(e):
            svec = jnp.zeros((_L,), _f32)
            for h in range(_H):
                prod = (qv[e, pl.ds(h * _HD, _HD)]
                        * kv[e, pl.ds(h * _HD, _HD)])
                pa, pb = plsc.unpack(prod, format=fmt)
                sh = jnp.sum(pa + pb) * _SCALE
                svec = jnp.where(iota == h, sh, svec)
            sbuf[pl.ds(e * _L, _L)] = svec
            mref[...] = jnp.maximum(mref[...], svec)

        pltpu.async_copy(sbuf, sc_hbm.at[pl.ds(cb * _L, _CHA * _L)], so)

    prefetch(0, 0, True)
    prefetch(1, 1, True)

    @pl.loop(0, _NCHA // 2)
    def _pair(g):
        c0 = 2 * g
        consume(c0, 0)

        @pl.when(g < _NCHA // 2 - 1)
        def _():
            prefetch(c0 + 2, 0, False)

        consume(c0 + 1, 1)

        @pl.when(g < _NCHA // 2 - 1)
        def _():
            prefetch(c0 + 3, 1, False)

    pltpu.make_async_copy(bufs[0][4], sc_hbm.at[pl.ds(base * _L, _CHA * _L)],
                          bufs[0][7]).wait()
    pltpu.make_async_copy(bufs[1][4], sc_hbm.at[pl.ds(base * _L, _CHA * _L)],
                          bufs[1][7]).wait()
    pltpu.sync_copy(mref, wmax_hbm.at[wid])


def _exp_body(wm_ref, sc_ref, ex_ref):
    M = jnp.max(wm_ref[...])
    ex_ref[...] = jnp.exp(sc_ref[...] - M)


def _exp_stage(scores, wmax):
    total = _E_PAD * _L
    blk = total // 10
    return pl.pallas_call(
        _exp_body,
        grid=(10,),
        in_specs=[
            pl.BlockSpec((_NW, _L), lambda i: (0, 0)),
            pl.BlockSpec((blk,), lambda i: (i,)),
        ],
        out_specs=pl.BlockSpec((blk,), lambda i: (i,)),
        out_shape=jax.ShapeDtypeStruct((total,), _f32),
    )(wmax, scores)


def _make_aggregate(p):
    aw = 80 if p == 0 else 64
    out_type = jax.ShapeDtypeStruct((2, _N_PAD, aw), _f32)
    scratch = []
    for _ in range(3):
        scratch += [
            pltpu.VMEM((_CH,), _i32),
            pltpu.VMEM((_CH,), _i32),
            pltpu.VMEM((_CH * _L,), _f32),
            pltpu.VMEM((_CH, 64), _f32),
            pltpu.VMEM((_CH, aw), _f32),
            pltpu.SemaphoreType.DMA,
            pltpu.SemaphoreType.DMA,
            pltpu.SemaphoreType.DMA,
        ]
    scratch += [
        pltpu.VMEM_SHARED((_N_PAD, aw), _f32),
    ]

    def body(sc_hbm, tgt_hbm, src_hbm, v4_hbm, acc_hbm, *scr):
        bufs = [scr[0:8], scr[8:16], scr[16:24]]
        acc_sh = scr[24]
        ci = lax.axis_index("c")
        sid = lax.axis_index("s")

        cols = [jnp.full((_L,), 2 * p + j // 2, _i32) + ci * 4
                for j in range(4)]

        z = jnp.zeros((_L,), _f32)
        mbuf0 = bufs[0][4]

        @pl.loop(0, _CH)
        def _zrow(r):
            for j in range(aw // _L):
                mbuf0[r, pl.ds(j * _L, _L)] = z

        done = 0
        while done < _ZROWS:
            step = min(_CH, _ZROWS - done)
            src = mbuf0 if step == _CH else mbuf0.at[pl.ds(0, step)]
            pltpu.sync_copy(src, acc_sh.at[pl.ds(sid * _ZROWS + done, step)])
            done += step
        plsc.subcore_barrier()

        base = sid * _EPC
        vq = v4_hbm.at[2 * ci + p]

        def prefetch(c, b, first):
            ti, si, sv, vv, mbuf, ss, sV, sM = bufs[b]
            if not first:
                pltpu.make_async_copy(mbuf, acc_sh.at[ti], sM).wait()
            cb = base + c * _CH
            pltpu.sync_copy(tgt_hbm.at[pl.ds(cb, _CH)], ti)
            pltpu.sync_copy(src_hbm.at[pl.ds(cb, _CH)], si)
            pltpu.async_copy(sc_hbm.at[pl.ds(cb * _L, _CH * _L)], sv, ss)
            pltpu.async_copy(vq.at[si], vv, sV)

        def consume(c, b):
            ti, si, sv, vv, mbuf, ss, sV, sM = bufs[b]
            cb = base + c * _CH
            pltpu.make_async_copy(sc_hbm.at[pl.ds(cb * _L, _CH * _L)],
                                  sv, ss).wait()
            pltpu.make_async_copy(vq.at[si], vv, sV).wait()

            @pl.loop(0, _CH, unroll=4)
            def _edge(e):
                e16 = e * _L
                if p == 0:
                    mbuf[e, pl.ds(64, _L)] = sv[pl.ds(e16, _L)]
                for j in range(4):
                    w = plsc.load_gather(sv, [cols[j] + e16])
                    mbuf[e, pl.ds(j * _L, _L)] = (
                        vv[e, pl.ds(j * _L, _L)] * w)

            pltpu.async_copy(mbuf, acc_sh.at[ti], sM, add=True)

        nch = _EPC // _CH
        prefetch(0, 0, True)
        prefetch(1, 1, True)
        prefetch(2, 2, True)

        @pl.loop(0, (nch + 2) // 3)
        def _trip(g):
            c = 3 * g
            consume(c, 0)

            @pl.when(c + 1 < nch)
            def _():
                consume(c + 1, 1)

            @pl.when(c + 3 < nch)
            def _():
                prefetch(c + 3, 0, False)

            @pl.when(c + 2 < nch)
            def _():
                consume(c + 2, 2)

            @pl.when(c + 4 < nch)
            def _():
                prefetch(c + 4, 1, False)

            @pl.when(c + 5 < nch)
            def _():
                prefetch(c + 5, 2, False)

        for b in range(3):
            ti, si, sv, vv, mbuf, ss, sV, sM = bufs[b]
            pltpu.make_async_copy(mbuf, acc_sh.at[ti], sM).wait()

        plsc.subcore_barrier()
        r0 = sid * _ZROWS
        pltpu.sync_copy(acc_sh.at[pl.ds(r0, _ZROWS)],
                        acc_hbm.at[ci].at[pl.ds(r0, _ZROWS)])

    return pl.kernel(body, out_type=out_type, mesh=_mesh,
                     compiler_params=_sc_params, scratch_types=scratch)


_edge_aggregate0 = _make_aggregate(0)
_edge_aggregate1 = _make_aggregate(1)


def _layernorm(x, g, b):
    mu = jnp.mean(x, axis=-1, keepdims=True)
    var = jnp.mean((x - mu) ** 2, axis=-1, keepdims=True)
    return (x - mu) / jnp.sqrt(var + 1e-5) * g + b


def _final_body(ns_ref, a0_ref, a1_ref, exp_ref, ow_ref, ob_ref,
                w1_ref, b1_ref, w2_ref, b2_ref, g1_ref, be1_ref, g2_ref,
                be2_ref, out_ref):
    num = jnp.concatenate([a0_ref[0][:, 0:64], a1_ref[0],
                           a0_ref[1][:, 0:64], a1_ref[1]], axis=1)
    den8 = a0_ref[0][:, 64:64 + _H]
    div = lax.dot_general(den8, exp_ref[...], (((1,), (0,)), ((), ())),
                          preferred_element_type=_f32) + 1e-16
    attn = num / div
    attn = lax.dot_general(attn, ow_ref[...], _C11,
                           preferred_element_type=_f32) + ob_ref[...]
    x = ns_ref[...] + attn
    x = _layernorm(x, g1_ref[...], be1_ref[...])
    h1 = lax.dot_general(x, w1_ref[...], _C11,
                         preferred_element_type=_f32) + b1_ref[...]
    h1 = jnp.maximum(h1, 0.0)
    ff = lax.dot_general(h1, w2_ref[...], _C11,
                         preferred_element_type=_f32) + b2_ref[...]
    x = x + ff
    out_ref[...] = _layernorm(x, g2_ref[...], be2_ref[...])


def _final_dense(ns, acc0, acc1, out_W, out_b, W1, b1, W2, b2,
                 g1, be1, g2, be2):
    blk = _N // 10
    full = lambda shape: pl.BlockSpec(shape, lambda i: tuple(0 for _ in shape))
    return pl.pallas_call(
        _final_body,
        grid=(10,),
        in_specs=[
            pl.BlockSpec((blk, _D), lambda i: (i, 0)),
            pl.BlockSpec((2, blk, 80), lambda i: (0, i, 0)),
            pl.BlockSpec((2, blk, 64), lambda i: (0, i, 0)),
            full((_H, _D)),
            full((_D, _D)),
            full((1, _D)),
            full((_FF, _D)),
            full((1, _FF)),
            full((_D, _FF)),
            full((1, _D)),
            full((1, _D)),
            full((1, _D)),
            full((1, _D)),
            full((1, _D)),
        ],
        out_specs=pl.BlockSpec((blk, _D), lambda i: (i, 0)),
        out_shape=jax.ShapeDtypeStruct((_N, _D), _f32),
    )(ns, acc0, acc1, jnp.asarray(_EXPAND), out_W, out_b, W1, b1, W2, b2,
      g1, be1, g2, be2)


def kernel(node_states, edge_lists, qkv_W, qkv_b, out_W, out_b,
           W1, b1, W2, b2, g1, be1, g2, be2):
    src = edge_lists[0].astype(_i32)
    tgt = edge_lists[1].astype(_i32)
    zpad = jnp.zeros((_E_PAD - _E,), _i32)
    spad = jnp.full((_E_PAD - _E,), _N_PAD - 1, _i32)
    src_p = jnp.concatenate([src, zpad])
    tga = jnp.concatenate([tgt, zpad])
    tgs = jnp.concatenate([tgt, spad])

    q, k, v4 = _qkv_proj(node_states, qkv_W, qkv_b.reshape(1, -1))
    scores, wmax = _edge_scores(q, k, tga, src_p)
    ex = _exp_stage(scores, wmax)
    acc0 = _edge_aggregate0(ex, tgs, src_p, v4)
    acc1 = _edge_aggregate1(ex, tgs, src_p, v4)
    return _final_dense(node_states, acc0, acc1,
                        out_W, out_b.reshape(1, -1),
                        W1, b1.reshape(1, -1), W2, b2.reshape(1, -1),
                        g1.reshape(1, -1), be1.reshape(1, -1),
                        g2.reshape(1, -1), be2.reshape(1, -1))


# speedup vs baseline: 4.5397x; 1.2175x over previous
import dataclasses
import functools

import jax
import jax.numpy as jnp
import numpy as np
from jax import lax
from jax.experimental import pallas as pl
from jax.experimental.pallas import tpu as pltpu
from jax.experimental.pallas import tpu_sc as plsc

_N = 10000
_E = 160000
_D = 256
_H = 8
_FF = 1024
_HD = _D // _H
_SCALE = float(_HD) ** -0.5

_NC, _NS, _L = 2, 16, 16
_NW = _NC * _NS
_CH = 128
_N_PAD = 10240
_E_PAD = 163840
_EPW = _E_PAD // _NW
_EPC = _E_PAD // _NS
_ZROWS = _N_PAD // _NS

_f32 = jnp.float32
_i32 = jnp.int32
_C11 = (((1,), (1,)), ((), ()))

_sc_params = pltpu.CompilerParams()
if "needs_layout_passes" in pltpu.CompilerParams.__dataclass_fields__:
    _sc_params = dataclasses.replace(_sc_params, needs_layout_passes=False)
if "use_tc_tiling_on_sc" in pltpu.CompilerParams.__dataclass_fields__:
    _sc_params = dataclasses.replace(_sc_params, use_tc_tiling_on_sc=False)

_mesh = plsc.VectorSubcoreMesh(
    core_axis_name="c", subcore_axis_name="s", num_cores=_NC, num_subcores=_NS
)

_EXPAND = np.repeat(np.eye(_H, dtype=np.float32), _HD, axis=1)

_VPERM = np.empty((_D,), np.int64)
for _g in range(_H):
    for _i in range(32):
        _VPERM[32 * _g + _i] = 32 * _g + (
            2 * _i if _i < 16 else 2 * (_i - 16) + 1)


def _qkv_body(ns_ref, w_ref, b_ref, q_ref, k_ref, v_ref):
    x = lax.dot_general(ns_ref[...], w_ref[...], _C11,
                        preferred_element_type=_f32) + b_ref[...]
    q_ref[...] = x[:, 0:_D].astype(jnp.bfloat16)
    k_ref[...] = x[:, _D:2 * _D].astype(jnp.bfloat16)
    for j in range(4):
        v_ref[j] = x[:, 2 * _D + 64 * j:2 * _D + 64 * (j + 1)].astype(
            jnp.bfloat16)


def _qkv_proj(ns, qkv_W, qkv_b):
    blk = _N // 5
    return pl.pallas_call(
        _qkv_body,
        grid=(5,),
        in_specs=[
            pl.BlockSpec((blk, _D), lambda i: (i, 0)),
            pl.BlockSpec((3 * _D, _D), lambda i: (0, 0)),
            pl.BlockSpec((1, 3 * _D), lambda i: (0, 0)),
        ],
        out_specs=[
            pl.BlockSpec((blk, _D), lambda i: (i, 0)),
            pl.BlockSpec((blk, _D), lambda i: (i, 0)),
            pl.BlockSpec((4, blk, 64), lambda i: (0, i, 0)),
        ],
        out_shape=[
            jax.ShapeDtypeStruct((_N, _D), jnp.bfloat16),
            jax.ShapeDtypeStruct((_N, _D), jnp.bfloat16),
            jax.ShapeDtypeStruct((4, _N, 64), jnp.bfloat16),
        ],
    )(ns, qkv_W, qkv_b)


_CHA = 128
_NCHA = _EPW // _CHA
_bf16 = jnp.bfloat16

_scores_scratch = []
for _ in range(2):
    _scores_scratch += [
        pltpu.VMEM((_CHA,), _i32),
        pltpu.VMEM((_CHA,), _i32),
        pltpu.VMEM((_CHA, _D), _bf16),
        pltpu.VMEM((_CHA, _D), _bf16),
        pltpu.VMEM((_CHA * _L,), _f32),
        pltpu.SemaphoreType.DMA,
        pltpu.SemaphoreType.DMA,
        pltpu.SemaphoreType.DMA,
    ]
_scores_scratch.append(pltpu.VMEM((_L,), _f32))


@functools.partial(
    pl.kernel,
    out_type=(
        jax.ShapeDtypeStruct((_E_PAD * _L,), _f32),
        jax.ShapeDtypeStruct((_NW, _L), _f32),
    ),
    mesh=_mesh,
    compiler_params=_sc_params,
    scratch_types=_scores_scratch,
)
def _edge_scores(q_hbm, k_hbm, tgt_hbm, src_hbm, sc_hbm, wmax_hbm, *scr):
    bufs = [scr[0:8], scr[8:16]]
    mref = scr[16]
    ci = lax.axis_index("c")
    sid = lax.axis_index("s")
    wid = sid * _NC + ci
    base = wid * _EPW
    iota = lax.iota(_i32, _L)
    fmt = plsc.PackFormat.INTERLEAVED
    mref[...] = jnp.zeros((_L,), _f32)

    def prefetch(c, b, first):
        ti, si, qv, kv, sbuf, sq, sk, so = bufs[b]
        if not first:
            pltpu.make_async_copy(sbuf, sc_hbm.at[pl.ds(base * _L, _CHA * _L)],
                                  so).wait()
        cb = base + c * _CHA
        pltpu.sync_copy(tgt_hbm.at[pl.ds(cb, _CHA)], ti)
        pltpu.sync_copy(src_hbm.at[pl.ds(cb, _CHA)], si)
        pltpu.async_copy(q_hbm.at[ti], qv, sq)
        pltpu.async_copy(k_hbm.at[si], kv, sk)

    def consume(c, b):
        ti, si, qv, kv, sbuf, sq, sk, so = bufs[b]
        cb = base + c * _CHA
        pltpu.make_async_copy(q_hbm.at[ti], qv, sq).wait()
        pltpu.make_async_copy(k_hbm.at[si], kv, sk).wait()

        @pl.loop(0, _CHA, unroll=4)
        def _edge(e):
            svec = jnp.zeros((_L,), _f32)
            for h in range(_H):
                prod = (qv[e, pl.ds(h * _HD, _HD)]
                        * kv[e, pl.ds(h * _HD, _HD)])
                pa, pb = plsc.unpack(prod, format=fmt)
                sh = jnp.sum(pa + pb) * _SCALE
                svec = jnp.where(iota == h, sh, svec)
            sbuf[pl.ds(e * _L, _L)] = svec
            mref[...] = jnp.maximum(mref[...], svec)

        pltpu.async_copy(sbuf, sc_hbm.at[pl.ds(cb * _L, _CHA * _L)], so)

    prefetch(0, 0, True)
    prefetch(1, 1, True)

    @pl.loop(0, _NCHA // 2)
    def _pair(g):
        c0 = 2 * g
        consume(c0, 0)

        @pl.when(g < _NCHA // 2 - 1)
        def _():
            prefetch(c0 + 2, 0, False)

        consume(c0 + 1, 1)

        @pl.when(g < _NCHA // 2 - 1)
        def _():
            prefetch(c0 + 3, 1, False)

    pltpu.make_async_copy(bufs[0][4], sc_hbm.at[pl.ds(base * _L, _CHA * _L)],
                          bufs[0][7]).wait()
    pltpu.make_async_copy(bufs[1][4], sc_hbm.at[pl.ds(base * _L, _CHA * _L)],
                          bufs[1][7]).wait()
    pltpu.sync_copy(mref, wmax_hbm.at[wid])


def _exp_body(wm_ref, sc_ref, ex_ref):
    M = jnp.max(wm_ref[...])
    ex_ref[...] = jnp.exp(sc_ref[...] - M)


def _exp_stage(scores, wmax):
    total = _E_PAD * _L
    blk = total // 10
    return pl.pallas_call(
        _exp_body,
        grid=(10,),
        in_specs=[
            pl.BlockSpec((_NW, _L), lambda i: (0, 0)),
            pl.BlockSpec((blk,), lambda i: (i,)),
        ],
        out_specs=pl.BlockSpec((blk,), lambda i: (i,)),
        out_shape=jax.ShapeDtypeStruct((total,), _f32),
    )(wmax, scores)


def _make_aggregate(p):
    aw = 80 if p == 0 else 64
    out_type = jax.ShapeDtypeStruct((2, _N_PAD, aw), _f32)
    scratch = []
    for _ in range(3):
        scratch += [
            pltpu.VMEM((_CH,), _i32),
            pltpu.VMEM((_CH,), _i32),
            pltpu.VMEM((_CH * _L,), _f32),
            pltpu.VMEM((_CH, 64), _bf16),
            pltpu.VMEM((_CH, aw), _f32),
            pltpu.SemaphoreType.DMA,
            pltpu.SemaphoreType.DMA,
            pltpu.SemaphoreType.DMA,
        ]
    scratch += [
        pltpu.VMEM_SHARED((_N_PAD, aw), _f32),
    ]

    def body(sc_hbm, tgt_hbm, src_hbm, v4_hbm, acc_hbm, *scr):
        bufs = [scr[0:8], scr[8:16], scr[16:24]]
        acc_sh = scr[24]
        ci = lax.axis_index("c")
        sid = lax.axis_index("s")

        cols = [jnp.full((_L,), 2 * p + g, _i32) + ci * 4 for g in range(2)]
        fmt = plsc.PackFormat.INTERLEAVED

        z = jnp.zeros((_L,), _f32)
        mbuf0 = bufs[0][4]

        @pl.loop(0, _CH)
        def _zrow(r):
            for j in range(aw // _L):
                mbuf0[r, pl.ds(j * _L, _L)] = z

        done = 0
        while done < _ZROWS:
            step = min(_CH, _ZROWS - done)
            src = mbuf0 if step == _CH else mbuf0.at[pl.ds(0, step)]
            pltpu.sync_copy(src, acc_sh.at[pl.ds(sid * _ZROWS + done, step)])
            done += step
        plsc.subcore_barrier()

        base = sid * _EPC
        vq = v4_hbm.at[2 * ci + p]

        def prefetch(c, b, first):
            ti, si, sv, vv, mbuf, ss, sV, sM = bufs[b]
            if not first:
                pltpu.make_async_copy(mbuf, acc_sh.at[ti], sM).wait()
            cb = base + c * _CH
            pltpu.sync_copy(tgt_hbm.at[pl.ds(cb, _CH)], ti)
            pltpu.sync_copy(src_hbm.at[pl.ds(cb, _CH)], si)
            pltpu.async_copy(sc_hbm.at[pl.ds(cb * _L, _CH * _L)], sv, ss)
            pltpu.async_copy(vq.at[si], vv, sV)

        def consume(c, b):
            ti, si, sv, vv, mbuf, ss, sV, sM = bufs[b]
            cb = base + c * _CH
            pltpu.make_async_copy(sc_hbm.at[pl.ds(cb * _L, _CH * _L)],
                                  sv, ss).wait()
            pltpu.make_async_copy(vq.at[si], vv, sV).wait()

            @pl.loop(0, _CH, unroll=4)
            def _edge(e):
                e16 = e * _L
                if p == 0:
                    mbuf[e, pl.ds(64, _L)] = sv[pl.ds(e16, _L)]
                for g in range(2):
                    va, vo = plsc.unpack(vv[e, pl.ds(32 * g, 32)], format=fmt)
                    w = plsc.load_gather(sv, [cols[g] + e16])
                    mbuf[e, pl.ds(32 * g, _L)] = va * w
                    mbuf[e, pl.ds(32 * g + _L, _L)] = vo * w

            pltpu.async_copy(mbuf, acc_sh.at[ti], sM, add=True)

        nch = _EPC // _CH
        prefetch(0, 0, True)
        prefetch(1, 1, True)
        prefetch(2, 2, True)

        @pl.loop(0, (nch + 2) // 3)
        def _trip(g):
            c = 3 * g
            consume(c, 0)

            @pl.when(c + 1 < nch)
            def _():
                consume(c + 1, 1)

            @pl.when(c + 3 < nch)
            def _():
                prefetch(c + 3, 0, False)

            @pl.when(c + 2 < nch)
            def _():
                consume(c + 2, 2)

            @pl.when(c + 4 < nch)
            def _():
                prefetch(c + 4, 1, False)

            @pl.when(c + 5 < nch)
            def _():
                prefetch(c + 5, 2, False)

        for b in range(3):
            ti, si, sv, vv, mbuf, ss, sV, sM = bufs[b]
            pltpu.make_async_copy(mbuf, acc_sh.at[ti], sM).wait()

        plsc.subcore_barrier()
        r0 = sid * _ZROWS
        pltpu.sync_copy(acc_sh.at[pl.ds(r0, _ZROWS)],
                        acc_hbm.at[ci].at[pl.ds(r0, _ZROWS)])

    return pl.kernel(body, out_type=out_type, mesh=_mesh,
                     compiler_params=_sc_params, scratch_types=scratch)


_edge_aggregate0 = _make_aggregate(0)
_edge_aggregate1 = _make_aggregate(1)


def _layernorm(x, g, b):
    mu = jnp.mean(x, axis=-1, keepdims=True)
    var = jnp.mean((x - mu) ** 2, axis=-1, keepdims=True)
    return (x - mu) / jnp.sqrt(var + 1e-5) * g + b


def _final_body(ns_ref, a0_ref, a1_ref, exp_ref, ow_ref, ob_ref,
                w1_ref, b1_ref, w2_ref, b2_ref, g1_ref, be1_ref, g2_ref,
                be2_ref, out_ref):
    num = jnp.concatenate([a0_ref[0][:, 0:64], a1_ref[0],
                           a0_ref[1][:, 0:64], a1_ref[1]], axis=1)
    den8 = a0_ref[0][:, 64:64 + _H]
    div = lax.dot_general(den8, exp_ref[...], (((1,), (0,)), ((), ())),
                          preferred_element_type=_f32) + 1e-16
    attn = num / div
    attn = lax.dot_general(attn, ow_ref[...], _C11,
                           preferred_element_type=_f32) + ob_ref[...]
    x = ns_ref[...] + attn
    x = _layernorm(x, g1_ref[...], be1_ref[...])
    h1 = lax.dot_general(x, w1_ref[...], _C11,
                         preferred_element_type=_f32) + b1_ref[...]
    h1 = jnp.maximum(h1, 0.0)
    ff = lax.dot_general(h1, w2_ref[...], _C11,
                         preferred_element_type=_f32) + b2_ref[...]
    x = x + ff
    out_ref[...] = _layernorm(x, g2_ref[...], be2_ref[...])


def _final_dense(ns, acc0, acc1, out_W, out_b, W1, b1, W2, b2,
                 g1, be1, g2, be2):
    blk = _N // 10
    full = lambda shape: pl.BlockSpec(shape, lambda i: tuple(0 for _ in shape))
    return pl.pallas_call(
        _final_body,
        grid=(10,),
        in_specs=[
            pl.BlockSpec((blk, _D), lambda i: (i, 0)),
            pl.BlockSpec((2, blk, 80), lambda i: (0, i, 0)),
            pl.BlockSpec((2, blk, 64), lambda i: (0, i, 0)),
            full((_H, _D)),
            full((_D, _D)),
            full((1, _D)),
            full((_FF, _D)),
            full((1, _FF)),
            full((_D, _FF)),
            full((1, _D)),
            full((1, _D)),
            full((1, _D)),
            full((1, _D)),
            full((1, _D)),
        ],
        out_specs=pl.BlockSpec((blk, _D), lambda i: (i, 0)),
        out_shape=jax.ShapeDtypeStruct((_N, _D), _f32),
    )(ns, acc0, acc1, jnp.asarray(_EXPAND), out_W, out_b, W1, b1, W2, b2,
      g1, be1, g2, be2)


def kernel(node_states, edge_lists, qkv_W, qkv_b, out_W, out_b,
           W1, b1, W2, b2, g1, be1, g2, be2):
    src = edge_lists[0].astype(_i32)
    tgt = edge_lists[1].astype(_i32)
    zpad = jnp.zeros((_E_PAD - _E,), _i32)
    spad = jnp.full((_E_PAD - _E,), _N_PAD - 1, _i32)
    src_p = jnp.concatenate([src, zpad])
    tga = jnp.concatenate([tgt, zpad])
    tgs = jnp.concatenate([tgt, spad])

    q, k, v4 = _qkv_proj(node_states, qkv_W, qkv_b.reshape(1, -1))
    scores, wmax = _edge_scores(q, k, tga, src_p)
    ex = _exp_stage(scores, wmax)
    acc0 = _edge_aggregate0(ex, tgs, src_p, v4)
    acc1 = _edge_aggregate1(ex, tgs, src_p, v4)
    return _final_dense(node_states, acc0, acc1,
                        out_W[:, jnp.asarray(_VPERM)], out_b.reshape(1, -1),
                        W1, b1.reshape(1, -1), W2, b2.reshape(1, -1),
                        g1.reshape(1, -1), be1.reshape(1, -1),
                        g2.reshape(1, -1), be2.reshape(1, -1))


# speedup vs baseline: 4.5445x; 1.0011x over previous
import dataclasses
import functools

import jax
import jax.numpy as jnp
import numpy as np
from jax import lax
from jax.experimental import pallas as pl
from jax.experimental.pallas import tpu as pltpu
from jax.experimental.pallas import tpu_sc as plsc

_N = 10000
_E = 160000
_D = 256
_H = 8
_FF = 1024
_HD = _D // _H
_SCALE = float(_HD) ** -0.5

_NC, _NS, _L = 2, 16, 16
_NW = _NC * _NS
_CH = 128
_N_PAD = 10240
_E_PAD = 163840
_EPW = _E_PAD // _NW
_EPC = _E_PAD // _NS
_ZROWS = _N_PAD // _NS

_f32 = jnp.float32
_i32 = jnp.int32
_C11 = (((1,), (1,)), ((), ()))

_sc_params = pltpu.CompilerParams()
if "needs_layout_passes" in pltpu.CompilerParams.__dataclass_fields__:
    _sc_params = dataclasses.replace(_sc_params, needs_layout_passes=False)
if "use_tc_tiling_on_sc" in pltpu.CompilerParams.__dataclass_fields__:
    _sc_params = dataclasses.replace(_sc_params, use_tc_tiling_on_sc=False)

_mesh = plsc.VectorSubcoreMesh(
    core_axis_name="c", subcore_axis_name="s", num_cores=_NC, num_subcores=_NS
)

_EXPAND = np.repeat(np.eye(_H, dtype=np.float32), _HD, axis=1)

_VPERM = np.empty((_D,), np.int64)
for _g in range(_H):
    for _i in range(32):
        _VPERM[32 * _g + _i] = 32 * _g + (
            2 * _i if _i < 16 else 2 * (_i - 16) + 1)


def _qkv_body(ns_ref, w_ref, b_ref, q_ref, k_ref, v_ref):
    x = lax.dot_general(ns_ref[...], w_ref[...], _C11,
                        preferred_element_type=_f32) + b_ref[...]
    q_ref[...] = x[:, 0:_D].astype(jnp.bfloat16)
    k_ref[...] = x[:, _D:2 * _D].astype(jnp.bfloat16)
    for j in range(4):
        v_ref[j] = x[:, 2 * _D + 64 * j:2 * _D + 64 * (j + 1)].astype(
            jnp.bfloat16)


def _qkv_proj(ns, qkv_W, qkv_b):
    blk = _N // 5
    return pl.pallas_call(
        _qkv_body,
        grid=(5,),
        in_specs=[
            pl.BlockSpec((blk, _D), lambda i: (i, 0)),
            pl.BlockSpec((3 * _D, _D), lambda i: (0, 0)),
            pl.BlockSpec((1, 3 * _D), lambda i: (0, 0)),
        ],
        out_specs=[
            pl.BlockSpec((blk, _D), lambda i: (i, 0)),
            pl.BlockSpec((blk, _D), lambda i: (i, 0)),
            pl.BlockSpec((4, blk, 64), lambda i: (0, i, 0)),
        ],
        out_shape=[
            jax.ShapeDtypeStruct((_N, _D), jnp.bfloat16),
            jax.ShapeDtypeStruct((_N, _D), jnp.bfloat16),
            jax.ShapeDtypeStruct((4, _N, 64), jnp.bfloat16),
        ],
    )(ns, qkv_W, qkv_b)


_CHA = 128
_NCHA = _EPW // _CHA
_bf16 = jnp.bfloat16

_scores_scratch = []
for _ in range(2):
    _scores_scratch += [
        pltpu.VMEM((_CHA,), _i32),
        pltpu.VMEM((_CHA,), _i32),
        pltpu.VMEM((_CHA, _D), _bf16),
        pltpu.VMEM((_CHA, _D), _bf16),
        pltpu.VMEM((_CHA * _L,), _f32),
        pltpu.SemaphoreType.DMA,
        pltpu.SemaphoreType.DMA,
        pltpu.SemaphoreType.DMA,
    ]
_scores_scratch.append(pltpu.VMEM((_L,), _f32))


@functools.partial(
    pl.kernel,
    out_type=(
        jax.ShapeDtypeStruct((_E_PAD * _L,), _f32),
        jax.ShapeDtypeStruct((_NW, _L), _f32),
    ),
    mesh=_mesh,
    compiler_params=_sc_params,
    scratch_types=_scores_scratch,
)
def _edge_scores(q_hbm, k_hbm, tgt_hbm, src_hbm, sc_hbm, wmax_hbm, *scr):
    bufs = [scr[0:8], scr[8:16]]
    mref = scr[16]
    ci = lax.axis_index("c")
    sid = lax.axis_index("s")
    wid = sid * _NC + ci
    base = wid * _EPW
    iota = lax.iota(_i32, _L)
    fmt = plsc.PackFormat.INTERLEAVED
    mref[...] = jnp.zeros((_L,), _f32)

    def prefetch(c, b, first):
        ti, si, qv, kv, sbuf, sq, sk, so = bufs[b]
        if not first:
            pltpu.make_async_copy(sbuf, sc_hbm.at[pl.ds(base * _L, _CHA * _L)],
                                  so).wait()
        cb = base + c * _CHA
        pltpu.sync_copy(tgt_hbm.at[pl.ds(cb, _CHA)], ti)
        pltpu.sync_copy(src_hbm.at[pl.ds(cb, _CHA)], si)
        pltpu.async_copy(q_hbm.at[ti], qv, sq)
        pltpu.async_copy(k_hbm.at[si], kv, sk)

    def consume(c, b):
        ti, si, qv, kv, sbuf, sq, sk, so = bufs[b]
        cb = base + c * _CHA
        pltpu.make_async_copy(q_hbm.at[ti], qv, sq).wait()
        pltpu.make_async_copy(k_hbm.at[si], kv, sk).wait()

        @pl.loop(0, _CHA, init_carry=mref[...], unroll=4)
        def _edge(e, m):
            svec = jnp.zeros((_L,), _f32)
            for h in range(_H):
                prod = (qv[e, pl.ds(h * _HD, _HD)]
                        * kv[e, pl.ds(h * _HD, _HD)])
                pa, pb = plsc.unpack(prod, format=fmt)
                sh = jnp.sum(pa + pb) * _SCALE
                svec = jnp.where(iota == h, sh, svec)
            sbuf[pl.ds(e * _L, _L)] = svec
            return jnp.maximum(m, svec)

        mref[...] = _edge
        pltpu.async_copy(sbuf, sc_hbm.at[pl.ds(cb * _L, _CHA * _L)], so)

    prefetch(0, 0, True)
    prefetch(1, 1, True)

    @pl.loop(0, _NCHA // 2)
    def _pair(g):
        c0 = 2 * g
        consume(c0, 0)

        @pl.when(g < _NCHA // 2 - 1)
        def _():
            prefetch(c0 + 2, 0, False)

        consume(c0 + 1, 1)

        @pl.when(g < _NCHA // 2 - 1)
        def _():
            prefetch(c0 + 3, 1, False)

    pltpu.make_async_copy(bufs[0][4], sc_hbm.at[pl.ds(base * _L, _CHA * _L)],
                          bufs[0][7]).wait()
    pltpu.make_async_copy(bufs[1][4], sc_hbm.at[pl.ds(base * _L, _CHA * _L)],
                          bufs[1][7]).wait()
    pltpu.sync_copy(mref, wmax_hbm.at[wid])


def _exp_body(wm_ref, sc_ref, ex_ref):
    M = jnp.max(wm_ref[...])
    ex_ref[...] = jnp.exp(sc_ref[...] - M)


def _exp_stage(scores, wmax):
    total = _E_PAD * _L
    blk = total // 10
    return pl.pallas_call(
        _exp_body,
        grid=(10,),
        in_specs=[
            pl.BlockSpec((_NW, _L), lambda i: (0, 0)),
            pl.BlockSpec((blk,), lambda i: (i,)),
        ],
        out_specs=pl.BlockSpec((blk,), lambda i: (i,)),
        out_shape=jax.ShapeDtypeStruct((total,), _f32),
    )(wmax, scores)


def _make_aggregate(p):
    aw = 80 if p == 0 else 64
    out_type = jax.ShapeDtypeStruct((2, _N_PAD, aw), _f32)
    scratch = []
    for _ in range(3):
        scratch += [
            pltpu.VMEM((_CH,), _i32),
            pltpu.VMEM((_CH,), _i32),
            pltpu.VMEM((_CH * _L,), _f32),
            pltpu.VMEM((_CH, 64), _bf16),
            pltpu.VMEM((_CH, aw), _f32),
            pltpu.SemaphoreType.DMA,
            pltpu.SemaphoreType.DMA,
            pltpu.SemaphoreType.DMA,
        ]
    scratch += [
        pltpu.VMEM_SHARED((_N_PAD, aw), _f32),
    ]

    def body(sc_hbm, tgt_hbm, src_hbm, v4_hbm, acc_hbm, *scr):
        bufs = [scr[0:8], scr[8:16], scr[16:24]]
        acc_sh = scr[24]
        ci = lax.axis_index("c")
        sid = lax.axis_index("s")

        cols = [jnp.full((_L,), 2 * p + g, _i32) + ci * 4 for g in range(2)]
        fmt = plsc.PackFormat.INTERLEAVED

        z = jnp.zeros((_L,), _f32)
        mbuf0 = bufs[0][4]

        @pl.loop(0, _CH)
        def _zrow(r):
            for j in range(aw // _L):
                mbuf0[r, pl.ds(j * _L, _L)] = z

        done = 0
        while done < _ZROWS:
            step = min(_CH, _ZROWS - done)
            src = mbuf0 if step == _CH else mbuf0.at[pl.ds(0, step)]
            pltpu.sync_copy(src, acc_sh.at[pl.ds(sid * _ZROWS + done, step)])
            done += step
        plsc.subcore_barrier()

        base = sid * _EPC
        vq = v4_hbm.at[2 * ci + p]

        def prefetch(c, b, first):
            ti, si, sv, vv, mbuf, ss, sV, sM = bufs[b]
            if not first:
                pltpu.make_async_copy(mbuf, acc_sh.at[ti], sM).wait()
            cb = base + c * _CH
            pltpu.sync_copy(tgt_hbm.at[pl.ds(cb, _CH)], ti)
            pltpu.sync_copy(src_hbm.at[pl.ds(cb, _CH)], si)
            pltpu.async_copy(sc_hbm.at[pl.ds(cb * _L, _CH * _L)], sv, ss)
            pltpu.async_copy(vq.at[si], vv, sV)

        def consume(c, b):
            ti, si, sv, vv, mbuf, ss, sV, sM = bufs[b]
            cb = base + c * _CH
            pltpu.make_async_copy(sc_hbm.at[pl.ds(cb * _L, _CH * _L)],
                                  sv, ss).wait()
            pltpu.make_async_copy(vq.at[si], vv, sV).wait()

            @pl.loop(0, _CH, unroll=4)
            def _edge(e):
                e16 = e * _L
                if p == 0:
                    mbuf[e, pl.ds(64, _L)] = sv[pl.ds(e16, _L)]
                for g in range(2):
                    va, vo = plsc.unpack(vv[e, pl.ds(32 * g, 32)], format=fmt)
                    w = plsc.load_gather(sv, [cols[g] + e16])
                    mbuf[e, pl.ds(32 * g, _L)] = va * w
                    mbuf[e, pl.ds(32 * g + _L, _L)] = vo * w

            pltpu.async_copy(mbuf, acc_sh.at[ti], sM, add=True)

        nch = _EPC // _CH
        prefetch(0, 0, True)
        prefetch(1, 1, True)
        prefetch(2, 2, True)

        @pl.loop(0, (nch + 2) // 3)
        def _trip(g):
            c = 3 * g
            consume(c, 0)

            @pl.when(c + 1 < nch)
            def _():
                consume(c + 1, 1)

            @pl.when(c + 3 < nch)
            def _():
                prefetch(c + 3, 0, False)

            @pl.when(c + 2 < nch)
            def _():
                consume(c + 2, 2)

            @pl.when(c + 4 < nch)
            def _():
                prefetch(c + 4, 1, False)

            @pl.when(c + 5 < nch)
            def _():
                prefetch(c + 5, 2, False)

        for b in range(3):
            ti, si, sv, vv, mbuf, ss, sV, sM = bufs[b]
            pltpu.make_async_copy(mbuf, acc_sh.at[ti], sM).wait()

        plsc.subcore_barrier()
        r0 = sid * _ZROWS
        pltpu.sync_copy(acc_sh.at[pl.ds(r0, _ZROWS)],
                        acc_hbm.at[ci].at[pl.ds(r0, _ZROWS)])

    return pl.kernel(body, out_type=out_type, mesh=_mesh,
                     compiler_params=_sc_params, scratch_types=scratch)


_edge_aggregate0 = _make_aggregate(0)
_edge_aggregate1 = _make_aggregate(1)


def _layernorm(x, g, b):
    mu = jnp.mean(x, axis=-1, keepdims=True)
    var = jnp.mean((x - mu) ** 2, axis=-1, keepdims=True)
    return (x - mu) / jnp.sqrt(var + 1e-5) * g + b


def _final_body(ns_ref, a0_ref, a1_ref, exp_ref, ow_ref, ob_ref,
                w1_ref, b1_ref, w2_ref, b2_ref, g1_ref, be1_ref, g2_ref,
                be2_ref, out_ref):
    num = jnp.concatenate([a0_ref[0][:, 0:64], a1_ref[0],
                           a0_ref[1][:, 0:64], a1_ref[1]], axis=1)
    den8 = a0_ref[0][:, 64:64 + _H]
    div = lax.dot_general(den8, exp_ref[...], (((1,), (0,)), ((), ())),
                          preferred_element_type=_f32) + 1e-16
    attn = num / div
    attn = lax.dot_general(attn, ow_ref[...], _C11,
                           preferred_element_type=_f32) + ob_ref[...]
    x = ns_ref[...] + attn
    x = _layernorm(x, g1_ref[...], be1_ref[...])
    h1 = lax.dot_general(x, w1_ref[...], _C11,
                         preferred_element_type=_f32) + b1_ref[...]
    h1 = jnp.maximum(h1, 0.0)
    ff = lax.dot_general(h1, w2_ref[...], _C11,
                         preferred_element_type=_f32) + b2_ref[...]
    x = x + ff
    out_ref[...] = _layernorm(x, g2_ref[...], be2_ref[...])


def _final_dense(ns, acc0, acc1, out_W, out_b, W1, b1, W2, b2,
                 g1, be1, g2, be2):
    blk = _N // 10
    full = lambda shape: pl.BlockSpec(shape, lambda i: tuple(0 for _ in shape))
    return pl.pallas_call(
        _final_body,
        grid=(10,),
        in_specs=[
            pl.BlockSpec((blk, _D), lambda i: (i, 0)),
            pl.BlockSpec((2, blk, 80), lambda i: (0, i, 0)),
            pl.BlockSpec((2, blk, 64), lambda i: (0, i, 0)),
            full((_H, _D)),
            full((_D, _D)),
            full((1, _D)),
            full((_FF, _D)),
            full((1, _FF)),
            full((_D, _FF)),
            full((1, _D)),
            full((1, _D)),
            full((1, _D)),
            full((1, _D)),
            full((1, _D)),
        ],
        out_specs=pl.BlockSpec((blk, _D), lambda i: (i, 0)),
        out_shape=jax.ShapeDtypeStruct((_N, _D), _f32),
    )(ns, acc0, acc1, jnp.asarray(_EXPAND), out_W, out_b, W1, b1, W2, b2,
      g1, be1, g2, be2)


def kernel(node_states, edge_lists, qkv_W, qkv_b, out_W, out_b,
           W1, b1, W2, b2, g1, be1, g2, be2):
    src = edge_lists[0].astype(_i32)
    tgt = edge_lists[1].astype(_i32)
    zpad = jnp.zeros((_E_PAD - _E,), _i32)
    spad = jnp.full((_E_PAD - _E,), _N_PAD - 1, _i32)
    src_p = jnp.concatenate([src, zpad])
    tga = jnp.concatenate([tgt, zpad])
    tgs = jnp.concatenate([tgt, spad])

    q, k, v4 = _qkv_proj(node_states, qkv_W, qkv_b.reshape(1, -1))
    scores, wmax = _edge_scores(q, k, tga, src_p)
    ex = _exp_stage(scores, wmax)
    acc0 = _edge_aggregate0(ex, tgs, src_p, v4)
    acc1 = _edge_aggregate1(ex, tgs, src_p, v4)
    return _final_dense(node_states, acc0, acc1,
                        out_W[:, jnp.asarray(_VPERM)], out_b.reshape(1, -1),
                        W1, b1.reshape(1, -1), W2, b2.reshape(1, -1),
                        g1.reshape(1, -1), be1.reshape(1, -1),
                        g2.reshape(1, -1), be2.reshape(1, -1))
